# baseline (speedup 1.0000x reference)
import numpy as np
import jax
import jax.numpy as jnp
from jax.experimental import pallas as pl
from jax.experimental.pallas import tpu as pltpu

NUM_NODES = 10000
NUM_EDGES = 50000
BATCH = 32
MEMORY_DIM = 128
TIME_DIM = 64
NUM_WALKS = 10
WALK_LEN = 3
TEMPERATURE = 0.1

E_ALL = 2 * NUM_EDGES
W_TOT = BATCH * NUM_WALKS
GROUPS = W_TOT // 8
CHUNK = 512
N_CHUNKS = -(-E_ALL // CHUNK)
E_PAD = N_CHUNKS * CHUNK

_TINY = np.float32(np.finfo(np.float32).tiny)
_NEG_INF = np.float32(-np.inf)


def _np_threefry_pair(k0, k1, x0, x1):
    x0 = np.asarray(x0, np.uint32).copy()
    x1 = np.asarray(x1, np.uint32).copy()
    ks0 = np.uint32(k0)
    ks1 = np.uint32(k1)
    ks2 = np.uint32(ks0 ^ ks1 ^ np.uint32(0x1BD11BDA))
    rots = ((13, 15, 26, 6), (17, 29, 16, 24))
    sched = ((ks1, ks2), (ks2, ks0), (ks0, ks1), (ks1, ks2), (ks2, ks0))
    x0 = (x0 + ks0).astype(np.uint32)
    x1 = (x1 + ks1).astype(np.uint32)
    for i in range(5):
        for r in rots[i % 2]:
            x0 = (x0 + x1).astype(np.uint32)
            x1 = ((x1 << np.uint32(r)) | (x1 >> np.uint32(32 - r))).astype(np.uint32)
            x1 = (x1 ^ x0).astype(np.uint32)
        a, b = sched[i]
        x0 = (x0 + a).astype(np.uint32)
        x1 = (x1 + b + np.uint32(i + 1)).astype(np.uint32)
    return x0, x1


def _derive_step_keys():
    keys = []
    k = (np.uint32(0), np.uint32(1234))
    for _ in range(WALK_LEN - 1):
        h0, h1 = _np_threefry_pair(k[0], k[1], np.zeros(2, np.uint32),
                                   np.arange(2, dtype=np.uint32))
        k = (h0[0], h1[0])
        keys.append((int(h0[1]), int(h1[1])))
    return keys


_STEP_KEYS = _derive_step_keys()


def _tf_bits(x1, k0, k1):
    ks0 = jnp.uint32(k0)
    ks1 = jnp.uint32(k1)
    ks2 = jnp.uint32(k0 ^ k1 ^ 0x1BD11BDA)
    rots = ((13, 15, 26, 6), (17, 29, 16, 24))
    sched = ((ks1, ks2), (ks2, ks0), (ks0, ks1), (ks1, ks2), (ks2, ks0))
    x0 = jnp.full_like(x1, ks0)
    x1 = x1 + ks1
    for i in range(5):
        for r in rots[i % 2]:
            x0 = x0 + x1
            x1 = (x1 << jnp.uint32(r)) | (x1 >> jnp.uint32(32 - r))
            x1 = x1 ^ x0
        a, b = sched[i]
        x0 = x0 + a
        x1 = x1 + b + jnp.uint32(i + 1)
    return x0 ^ x1


def _gumbel_from_bits(bits):
    fb = (bits >> jnp.uint32(9)) | jnp.uint32(0x3F800000)
    f = jax.lax.bitcast_convert_type(fb, jnp.float32) - jnp.float32(1.0)
    u = jnp.maximum(_TINY, f * (jnp.float32(1.0) - _TINY) + _TINY)
    return -jnp.log(-jnp.log(u))


def _walks_kernel(src_ref, dst_ref, t_ref, n0_ref, t0_ref,
                  on_ref, ot_ref, om_ref):
    g = pl.program_id(0)
    cn = n0_ref[0]
    ct = t0_ref[0]
    alive = jnp.ones((8, 1), dtype=jnp.bool_)
    wk = jax.lax.broadcasted_iota(jnp.int32, (8, 1), 0) + 8 * g

    on_ref[0, :, 0:1] = cn
    ot_ref[0, :, 0:1] = ct
    om_ref[0, :, 0:1] = jnp.ones((8, 1), jnp.float32)

    for s in range(WALK_LEN - 1):
        k0, k1 = _STEP_KEYS[s]

        def p1_body(i, tmax):
            src = src_ref[:, pl.ds(i * CHUNK, CHUNK)]
            tt = t_ref[:, pl.ds(i * CHUNK, CHUNK)]
            valid = (src == cn) & (tt < ct)
            tc = jnp.where(valid, tt, _NEG_INF)
            return jnp.maximum(tmax, jnp.max(tc, axis=1, keepdims=True))

        tmax_raw = jax.lax.fori_loop(
            0, N_CHUNKS, p1_body, jnp.full((8, 1), _NEG_INF, jnp.float32))
        has_valid = tmax_raw > _NEG_INF
        tmax = jnp.where(has_valid, tmax_raw, jnp.float32(0.0))

        def p2_body(i, carry):
            bv, bd, bt = carry
            src = src_ref[:, pl.ds(i * CHUNK, CHUNK)]
            dst = dst_ref[:, pl.ds(i * CHUNK, CHUNK)]
            tt = t_ref[:, pl.ds(i * CHUNK, CHUNK)]
            valid = (src == cn) & (tt < ct)
            j = i * CHUNK + jax.lax.broadcasted_iota(jnp.int32, (1, CHUNK), 1)
            idxs = (wk * E_ALL + j).astype(jnp.uint32)
            gmb = _gumbel_from_bits(_tf_bits(idxs, k0, k1))
            logits = jnp.where(valid, (tt - tmax) / jnp.float32(TEMPERATURE),
                               _NEG_INF)
            score = logits + gmb
            cmax = jnp.max(score, axis=1, keepdims=True)
            jbig = jnp.where(score == cmax, j, jnp.int32(2**31 - 1))
            jsel = jnp.min(jbig, axis=1, keepdims=True)
            onehot = j == jsel
            dsel = jnp.sum(jnp.where(onehot, dst, 0), axis=1, keepdims=True)
            tsel = jnp.sum(jnp.where(onehot, tt, jnp.float32(0.0)),
                           axis=1, keepdims=True)
            upd = cmax > bv
            return (jnp.where(upd, cmax, bv),
                    jnp.where(upd, dsel, bd),
                    jnp.where(upd, tsel, bt))

        bv, bd, bt = jax.lax.fori_loop(
            0, N_CHUNKS, p2_body,
            (jnp.full((8, 1), _NEG_INF, jnp.float32),
             jnp.zeros((8, 1), jnp.int32),
             jnp.zeros((8, 1), jnp.float32)))

        alive = alive & has_valid
        cn = jnp.where(alive, bd, cn)
        ct = jnp.where(alive, bt, ct)
        on_ref[0, :, s + 1:s + 2] = jnp.where(alive, bd, 0)
        ot_ref[0, :, s + 1:s + 2] = jnp.where(alive, bt, jnp.float32(0.0))
        om_ref[0, :, s + 1:s + 2] = alive.astype(jnp.float32)


def _restart_kernel(sn_ref, ct_ref, mem_ref, w_ref, b_ref, tw_ref, tb_ref,
                    out_ref, mrows_ref):
    def gather_body(i, _):
        idx = sn_ref[i]
        mrows_ref[pl.ds(i, 1), :] = mem_ref[pl.ds(idx, 1), :]
        return 0

    jax.lax.fori_loop(0, BATCH, gather_body, 0)
    mem = mrows_ref[...]
    te = jnp.cos(ct_ref[...] * tw_ref[...] + tb_ref[...])
    wm = w_ref[:, :MEMORY_DIM]
    wt = w_ref[:, MEMORY_DIM:]
    r = (jnp.sum(mem * wm, axis=1, keepdims=True)
         + jnp.sum(te * wt, axis=1, keepdims=True) + b_ref[...])
    out_ref[...] = jax.nn.sigmoid(r)


def kernel(source_nodes, current_times, edge_index, edge_time, memory_states,
           W_restart, b_restart, time_w, time_b):
    src_all = jnp.concatenate([edge_index[0], edge_index[1]]).astype(jnp.int32)
    dst_all = jnp.concatenate([edge_index[1], edge_index[0]]).astype(jnp.int32)
    t_all = jnp.concatenate([edge_time, edge_time]).astype(jnp.float32)
    pad = E_PAD - E_ALL
    src_p = jnp.pad(src_all, (0, pad), constant_values=-1)[None, :]
    dst_p = jnp.pad(dst_all, (0, pad), constant_values=0)[None, :]
    t_p = jnp.pad(t_all, (0, pad), constant_values=0.0)[None, :]

    n0 = jnp.broadcast_to(source_nodes.astype(jnp.int32)[:, None],
                          (BATCH, NUM_WALKS)).reshape(GROUPS, 8, 1)
    t0 = jnp.broadcast_to(current_times.astype(jnp.float32)[:, None],
                          (BATCH, NUM_WALKS)).reshape(GROUPS, 8, 1)

    full = pl.BlockSpec((1, E_PAD), lambda g: (0, 0))
    state = pl.BlockSpec((1, 8, 1), lambda g: (g, 0, 0))
    out3 = pl.BlockSpec((1, 8, WALK_LEN), lambda g: (g, 0, 0))

    on, ot, om = pl.pallas_call(
        _walks_kernel,
        grid=(GROUPS,),
        in_specs=[full, full, full, state, state],
        out_specs=[out3, out3, out3],
        out_shape=[
            jax.ShapeDtypeStruct((GROUPS, 8, WALK_LEN), jnp.int32),
            jax.ShapeDtypeStruct((GROUPS, 8, WALK_LEN), jnp.float32),
            jax.ShapeDtypeStruct((GROUPS, 8, WALK_LEN), jnp.float32),
        ],
    )(src_p, dst_p, t_p, n0, t0)

    walk_nodes = on.reshape(BATCH, NUM_WALKS, WALK_LEN)
    walk_times = ot.reshape(BATCH, NUM_WALKS, WALK_LEN)
    walk_masks = om.reshape(BATCH, NUM_WALKS, WALK_LEN)

    restart_probs = pl.pallas_call(
        _restart_kernel,
        in_specs=[
            pl.BlockSpec(memory_space=pltpu.SMEM),
            pl.BlockSpec((BATCH, 1), lambda: (0, 0)),
            pl.BlockSpec((NUM_NODES, MEMORY_DIM), lambda: (0, 0)),
            pl.BlockSpec((1, MEMORY_DIM + TIME_DIM), lambda: (0, 0)),
            pl.BlockSpec((1, 1), lambda: (0, 0)),
            pl.BlockSpec((1, TIME_DIM), lambda: (0, 0)),
            pl.BlockSpec((1, TIME_DIM), lambda: (0, 0)),
        ],
        out_specs=pl.BlockSpec((BATCH, 1), lambda: (0, 0)),
        out_shape=jax.ShapeDtypeStruct((BATCH, 1), jnp.float32),
        scratch_shapes=[pltpu.VMEM((BATCH, MEMORY_DIM), jnp.float32)],
    )(source_nodes.astype(jnp.int32),
      current_times.astype(jnp.float32)[:, None],
      memory_states.astype(jnp.float32),
      W_restart.astype(jnp.float32).reshape(1, -1),
      b_restart.astype(jnp.float32).reshape(1, 1),
      time_w.astype(jnp.float32)[None, :],
      time_b.astype(jnp.float32)[None, :])

    return walk_nodes, walk_times, walk_masks, restart_probs


# speedup vs baseline: 2.9334x; 2.9334x over previous
import numpy as np
import jax
import jax.numpy as jnp
from jax.experimental import pallas as pl
from jax.experimental.pallas import tpu as pltpu

NUM_NODES = 10000
NUM_EDGES = 50000
BATCH = 32
MEMORY_DIM = 128
TIME_DIM = 64
NUM_WALKS = 10
WALK_LEN = 3
TEMPERATURE = 0.1

E_ALL = 2 * NUM_EDGES
W_TOT = BATCH * NUM_WALKS
GROUPS = W_TOT // 8
CHUNK = 2048
N_CHUNKS = -(-E_ALL // CHUNK)
E_PAD = N_CHUNKS * CHUNK

_TINY = np.float32(np.finfo(np.float32).tiny)
_NEG_INF = np.float32(-np.inf)


def _np_threefry_pair(k0, k1, x0, x1):
    x0 = np.asarray(x0, np.uint32).copy()
    x1 = np.asarray(x1, np.uint32).copy()
    ks0 = np.uint32(k0)
    ks1 = np.uint32(k1)
    ks2 = np.uint32(ks0 ^ ks1 ^ np.uint32(0x1BD11BDA))
    rots = ((13, 15, 26, 6), (17, 29, 16, 24))
    sched = ((ks1, ks2), (ks2, ks0), (ks0, ks1), (ks1, ks2), (ks2, ks0))
    x0 = (x0 + ks0).astype(np.uint32)
    x1 = (x1 + ks1).astype(np.uint32)
    for i in range(5):
        for r in rots[i % 2]:
            x0 = (x0 + x1).astype(np.uint32)
            x1 = ((x1 << np.uint32(r)) | (x1 >> np.uint32(32 - r))).astype(np.uint32)
            x1 = (x1 ^ x0).astype(np.uint32)
        a, b = sched[i]
        x0 = (x0 + a).astype(np.uint32)
        x1 = (x1 + b + np.uint32(i + 1)).astype(np.uint32)
    return x0, x1


def _derive_step_keys():
    keys = []
    k = (np.uint32(0), np.uint32(1234))
    for _ in range(WALK_LEN - 1):
        h0, h1 = _np_threefry_pair(k[0], k[1], np.zeros(2, np.uint32),
                                   np.arange(2, dtype=np.uint32))
        k = (h0[0], h1[0])
        keys.append((int(h0[1]), int(h1[1])))
    return keys


_STEP_KEYS = _derive_step_keys()


def _tf_bits(x1, k0, k1):
    ks0 = jnp.uint32(k0)
    ks1 = jnp.uint32(k1)
    ks2 = jnp.uint32(k0 ^ k1 ^ 0x1BD11BDA)
    rots = ((13, 15, 26, 6), (17, 29, 16, 24))
    sched = ((ks1, ks2), (ks2, ks0), (ks0, ks1), (ks1, ks2), (ks2, ks0))
    x0 = jnp.full_like(x1, ks0)
    x1 = x1 + ks1
    for i in range(5):
        for r in rots[i % 2]:
            x0 = x0 + x1
            x1 = (x1 << jnp.uint32(r)) | (x1 >> jnp.uint32(32 - r))
            x1 = x1 ^ x0
        a, b = sched[i]
        x0 = x0 + a
        x1 = x1 + b + jnp.uint32(i + 1)
    return x0 ^ x1


def _gumbel_from_bits(bits):
    fb = (bits >> jnp.uint32(9)) | jnp.uint32(0x3F800000)
    f = jax.lax.bitcast_convert_type(fb, jnp.float32) - jnp.float32(1.0)
    u = jnp.maximum(_TINY, f * (jnp.float32(1.0) - _TINY) + _TINY)
    return -jnp.log(-jnp.log(u))


def _walks_kernel(src_ref, dst_ref, t_ref, n0_ref, t0_ref,
                  on_ref, ot_ref, om_ref):
    g = pl.program_id(0)
    cn = n0_ref[0]
    ct = t0_ref[0]
    alive = jnp.ones((8, 1), dtype=jnp.bool_)
    wk = jax.lax.broadcasted_iota(jnp.int32, (8, 1), 0) + 8 * g

    on_ref[0, :, 0:1] = cn
    ot_ref[0, :, 0:1] = ct
    om_ref[0, :, 0:1] = jnp.ones((8, 1), jnp.float32)

    for s in range(WALK_LEN - 1):
        k0, k1 = _STEP_KEYS[s]

        def p1_body(i, tmax):
            src = src_ref[:, pl.ds(i * CHUNK, CHUNK)]
            tt = t_ref[:, pl.ds(i * CHUNK, CHUNK)]
            valid = (src == cn) & (tt < ct)
            tc = jnp.where(valid, tt, _NEG_INF)
            return jnp.maximum(tmax, jnp.max(tc, axis=1, keepdims=True))

        tmax_raw = jax.lax.fori_loop(
            0, N_CHUNKS, p1_body, jnp.full((8, 1), _NEG_INF, jnp.float32))
        has_valid = tmax_raw > _NEG_INF
        tmax = jnp.where(has_valid, tmax_raw, jnp.float32(0.0))

        def p2_body(i, carry):
            bv, bd, bt = carry
            src = src_ref[:, pl.ds(i * CHUNK, CHUNK)]
            dst = dst_ref[:, pl.ds(i * CHUNK, CHUNK)]
            tt = t_ref[:, pl.ds(i * CHUNK, CHUNK)]
            valid = (src == cn) & (tt < ct)
            j = i * CHUNK + jax.lax.broadcasted_iota(jnp.int32, (1, CHUNK), 1)
            idxs = (wk * E_ALL + j).astype(jnp.uint32)
            gmb = _gumbel_from_bits(_tf_bits(idxs, k0, k1))
            logits = jnp.where(valid, (tt - tmax) / jnp.float32(TEMPERATURE),
                               _NEG_INF)
            score = logits + gmb
            cmax = jnp.max(score, axis=1, keepdims=True)
            jbig = jnp.where(score == cmax, j, jnp.int32(2**31 - 1))
            jsel = jnp.min(jbig, axis=1, keepdims=True)
            onehot = j == jsel
            dsel = jnp.sum(jnp.where(onehot, dst, 0), axis=1, keepdims=True)
            tsel = jnp.sum(jnp.where(onehot, tt, jnp.float32(0.0)),
                           axis=1, keepdims=True)
            upd = cmax > bv
            return (jnp.where(upd, cmax, bv),
                    jnp.where(upd, dsel, bd),
                    jnp.where(upd, tsel, bt))

        bv, bd, bt = jax.lax.fori_loop(
            0, N_CHUNKS, p2_body,
            (jnp.full((8, 1), _NEG_INF, jnp.float32),
             jnp.zeros((8, 1), jnp.int32),
             jnp.zeros((8, 1), jnp.float32)))

        alive = alive & has_valid
        cn = jnp.where(alive, bd, cn)
        ct = jnp.where(alive, bt, ct)
        on_ref[0, :, s + 1:s + 2] = jnp.where(alive, bd, 0)
        ot_ref[0, :, s + 1:s + 2] = jnp.where(alive, bt, jnp.float32(0.0))
        om_ref[0, :, s + 1:s + 2] = alive.astype(jnp.float32)


def _restart_kernel(sn_ref, ct_ref, mem_ref, w_ref, b_ref, tw_ref, tb_ref,
                    out_ref, mrows_ref):
    def gather_body(i, _):
        idx = sn_ref[i]
        mrows_ref[pl.ds(i, 1), :] = mem_ref[pl.ds(idx, 1), :]
        return 0

    jax.lax.fori_loop(0, BATCH, gather_body, 0)
    mem = mrows_ref[...]
    te = jnp.cos(ct_ref[...] * tw_ref[...] + tb_ref[...])
    wm = w_ref[:, :MEMORY_DIM]
    wt = w_ref[:, MEMORY_DIM:]
    r = (jnp.sum(mem * wm, axis=1, keepdims=True)
         + jnp.sum(te * wt, axis=1, keepdims=True) + b_ref[...])
    out_ref[...] = jax.nn.sigmoid(r)


def kernel(source_nodes, current_times, edge_index, edge_time, memory_states,
           W_restart, b_restart, time_w, time_b):
    src_all = jnp.concatenate([edge_index[0], edge_index[1]]).astype(jnp.int32)
    dst_all = jnp.concatenate([edge_index[1], edge_index[0]]).astype(jnp.int32)
    t_all = jnp.concatenate([edge_time, edge_time]).astype(jnp.float32)
    pad = E_PAD - E_ALL
    src_p = jnp.pad(src_all, (0, pad), constant_values=-1)[None, :]
    dst_p = jnp.pad(dst_all, (0, pad), constant_values=0)[None, :]
    t_p = jnp.pad(t_all, (0, pad), constant_values=0.0)[None, :]

    n0 = jnp.broadcast_to(source_nodes.astype(jnp.int32)[:, None],
                          (BATCH, NUM_WALKS)).reshape(GROUPS, 8, 1)
    t0 = jnp.broadcast_to(current_times.astype(jnp.float32)[:, None],
                          (BATCH, NUM_WALKS)).reshape(GROUPS, 8, 1)

    full = pl.BlockSpec((1, E_PAD), lambda g: (0, 0))
    state = pl.BlockSpec((1, 8, 1), lambda g: (g, 0, 0))
    out3 = pl.BlockSpec((1, 8, WALK_LEN), lambda g: (g, 0, 0))

    on, ot, om = pl.pallas_call(
        _walks_kernel,
        grid=(GROUPS,),
        in_specs=[full, full, full, state, state],
        out_specs=[out3, out3, out3],
        out_shape=[
            jax.ShapeDtypeStruct((GROUPS, 8, WALK_LEN), jnp.int32),
            jax.ShapeDtypeStruct((GROUPS, 8, WALK_LEN), jnp.float32),
            jax.ShapeDtypeStruct((GROUPS, 8, WALK_LEN), jnp.float32),
        ],
    )(src_p, dst_p, t_p, n0, t0)

    walk_nodes = on.reshape(BATCH, NUM_WALKS, WALK_LEN)
    walk_times = ot.reshape(BATCH, NUM_WALKS, WALK_LEN)
    walk_masks = om.reshape(BATCH, NUM_WALKS, WALK_LEN)

    restart_probs = pl.pallas_call(
        _restart_kernel,
        in_specs=[
            pl.BlockSpec(memory_space=pltpu.SMEM),
            pl.BlockSpec((BATCH, 1), lambda: (0, 0)),
            pl.BlockSpec((NUM_NODES, MEMORY_DIM), lambda: (0, 0)),
            pl.BlockSpec((1, MEMORY_DIM + TIME_DIM), lambda: (0, 0)),
            pl.BlockSpec((1, 1), lambda: (0, 0)),
            pl.BlockSpec((1, TIME_DIM), lambda: (0, 0)),
            pl.BlockSpec((1, TIME_DIM), lambda: (0, 0)),
        ],
        out_specs=pl.BlockSpec((BATCH, 1), lambda: (0, 0)),
        out_shape=jax.ShapeDtypeStruct((BATCH, 1), jnp.float32),
        scratch_shapes=[pltpu.VMEM((BATCH, MEMORY_DIM), jnp.float32)],
    )(source_nodes.astype(jnp.int32),
      current_times.astype(jnp.float32)[:, None],
      memory_states.astype(jnp.float32),
      W_restart.astype(jnp.float32).reshape(1, -1),
      b_restart.astype(jnp.float32).reshape(1, 1),
      time_w.astype(jnp.float32)[None, :],
      time_b.astype(jnp.float32)[None, :])

    return walk_nodes, walk_times, walk_masks, restart_probs


# speedup vs baseline: 3.7171x; 1.2671x over previous
import numpy as np
import jax
import jax.numpy as jnp
from jax.experimental import pallas as pl
from jax.experimental.pallas import tpu as pltpu

NUM_NODES = 10000
NUM_EDGES = 50000
BATCH = 32
MEMORY_DIM = 128
TIME_DIM = 64
NUM_WALKS = 10
WALK_LEN = 3
TEMPERATURE = 0.1

E_ALL = 2 * NUM_EDGES
W_TOT = BATCH * NUM_WALKS
GROUPS = W_TOT // 8
CHUNK = 4096
N_CHUNKS = -(-E_ALL // CHUNK)
E_PAD = N_CHUNKS * CHUNK

_TINY = np.float32(np.finfo(np.float32).tiny)
_NEG_INF = np.float32(-np.inf)


def _np_threefry_pair(k0, k1, x0, x1):
    x0 = np.asarray(x0, np.uint32).copy()
    x1 = np.asarray(x1, np.uint32).copy()
    ks0 = np.uint32(k0)
    ks1 = np.uint32(k1)
    ks2 = np.uint32(ks0 ^ ks1 ^ np.uint32(0x1BD11BDA))
    rots = ((13, 15, 26, 6), (17, 29, 16, 24))
    sched = ((ks1, ks2), (ks2, ks0), (ks0, ks1), (ks1, ks2), (ks2, ks0))
    x0 = (x0 + ks0).astype(np.uint32)
    x1 = (x1 + ks1).astype(np.uint32)
    for i in range(5):
        for r in rots[i % 2]:
            x0 = (x0 + x1).astype(np.uint32)
            x1 = ((x1 << np.uint32(r)) | (x1 >> np.uint32(32 - r))).astype(np.uint32)
            x1 = (x1 ^ x0).astype(np.uint32)
        a, b = sched[i]
        x0 = (x0 + a).astype(np.uint32)
        x1 = (x1 + b + np.uint32(i + 1)).astype(np.uint32)
    return x0, x1


def _derive_step_keys():
    keys = []
    k = (np.uint32(0), np.uint32(1234))
    for _ in range(WALK_LEN - 1):
        h0, h1 = _np_threefry_pair(k[0], k[1], np.zeros(2, np.uint32),
                                   np.arange(2, dtype=np.uint32))
        k = (h0[0], h1[0])
        keys.append((int(h0[1]), int(h1[1])))
    return keys


_STEP_KEYS = _derive_step_keys()


def _tf_bits(x1, k0, k1):
    ks0 = jnp.uint32(k0)
    ks1 = jnp.uint32(k1)
    ks2 = jnp.uint32(k0 ^ k1 ^ 0x1BD11BDA)
    rots = ((13, 15, 26, 6), (17, 29, 16, 24))
    sched = ((ks1, ks2), (ks2, ks0), (ks0, ks1), (ks1, ks2), (ks2, ks0))
    x0 = jnp.full_like(x1, ks0)
    x1 = x1 + ks1
    for i in range(5):
        for r in rots[i % 2]:
            x0 = x0 + x1
            x1 = (x1 << jnp.uint32(r)) | (x1 >> jnp.uint32(32 - r))
            x1 = x1 ^ x0
        a, b = sched[i]
        x0 = x0 + a
        x1 = x1 + b + jnp.uint32(i + 1)
    return x0 ^ x1


def _gumbel_from_bits(bits):
    fb = (bits >> jnp.uint32(9)) | jnp.uint32(0x3F800000)
    f = jax.lax.bitcast_convert_type(fb, jnp.float32) - jnp.float32(1.0)
    u = jnp.maximum(_TINY, f * (jnp.float32(1.0) - _TINY) + _TINY)
    return -jnp.log(-jnp.log(u))


def _walks_kernel(src_ref, dst_ref, t_ref, n0_ref, t0_ref,
                  on_ref, ot_ref, om_ref):
    g = pl.program_id(0)
    cn = n0_ref[0]
    ct = t0_ref[0]
    alive = jnp.ones((8, 1), dtype=jnp.bool_)
    wk = jax.lax.broadcasted_iota(jnp.int32, (8, 1), 0) + 8 * g

    on_ref[0, :, 0:1] = cn
    ot_ref[0, :, 0:1] = ct
    om_ref[0, :, 0:1] = jnp.ones((8, 1), jnp.float32)

    for s in range(WALK_LEN - 1):
        k0, k1 = _STEP_KEYS[s]

        def p1_body(i, tmax):
            src = src_ref[:, pl.ds(i * CHUNK, CHUNK)]
            tt = t_ref[:, pl.ds(i * CHUNK, CHUNK)]
            valid = (src == cn) & (tt < ct)
            tc = jnp.where(valid, tt, _NEG_INF)
            return jnp.maximum(tmax, jnp.max(tc, axis=1, keepdims=True))

        tmax_raw = jax.lax.fori_loop(
            0, N_CHUNKS, p1_body, jnp.full((8, 1), _NEG_INF, jnp.float32))
        has_valid = tmax_raw > _NEG_INF
        tmax = jnp.where(has_valid, tmax_raw, jnp.float32(0.0))

        def p2_body(i, carry):
            bv, bd, bt = carry
            src = src_ref[:, pl.ds(i * CHUNK, CHUNK)]
            dst = dst_ref[:, pl.ds(i * CHUNK, CHUNK)]
            tt = t_ref[:, pl.ds(i * CHUNK, CHUNK)]
            valid = (src == cn) & (tt < ct)
            j = i * CHUNK + jax.lax.broadcasted_iota(jnp.int32, (1, CHUNK), 1)
            idxs = (wk * E_ALL + j).astype(jnp.uint32)
            gmb = _gumbel_from_bits(_tf_bits(idxs, k0, k1))
            logits = jnp.where(valid, (tt - tmax) / jnp.float32(TEMPERATURE),
                               _NEG_INF)
            score = logits + gmb
            cmax = jnp.max(score, axis=1, keepdims=True)
            jbig = jnp.where(score == cmax, j, jnp.int32(2**31 - 1))
            jsel = jnp.min(jbig, axis=1, keepdims=True)
            onehot = j == jsel
            dsel = jnp.sum(jnp.where(onehot, dst, 0), axis=1, keepdims=True)
            tsel = jnp.sum(jnp.where(onehot, tt, jnp.float32(0.0)),
                           axis=1, keepdims=True)
            upd = cmax > bv
            return (jnp.where(upd, cmax, bv),
                    jnp.where(upd, dsel, bd),
                    jnp.where(upd, tsel, bt))

        bv, bd, bt = jax.lax.fori_loop(
            0, N_CHUNKS, p2_body,
            (jnp.full((8, 1), _NEG_INF, jnp.float32),
             jnp.zeros((8, 1), jnp.int32),
             jnp.zeros((8, 1), jnp.float32)))

        alive = alive & has_valid
        cn = jnp.where(alive, bd, cn)
        ct = jnp.where(alive, bt, ct)
        on_ref[0, :, s + 1:s + 2] = jnp.where(alive, bd, 0)
        ot_ref[0, :, s + 1:s + 2] = jnp.where(alive, bt, jnp.float32(0.0))
        om_ref[0, :, s + 1:s + 2] = alive.astype(jnp.float32)


def _restart_kernel(sn_ref, ct_ref, mem_ref, w_ref, b_ref, tw_ref, tb_ref,
                    out_ref, mrows_ref):
    def gather_body(i, _):
        idx = sn_ref[i]
        mrows_ref[pl.ds(i, 1), :] = mem_ref[pl.ds(idx, 1), :]
        return 0

    jax.lax.fori_loop(0, BATCH, gather_body, 0)
    mem = mrows_ref[...]
    te = jnp.cos(ct_ref[...] * tw_ref[...] + tb_ref[...])
    wm = w_ref[:, :MEMORY_DIM]
    wt = w_ref[:, MEMORY_DIM:]
    r = (jnp.sum(mem * wm, axis=1, keepdims=True)
         + jnp.sum(te * wt, axis=1, keepdims=True) + b_ref[...])
    out_ref[...] = jax.nn.sigmoid(r)


def kernel(source_nodes, current_times, edge_index, edge_time, memory_states,
           W_restart, b_restart, time_w, time_b):
    src_all = jnp.concatenate([edge_index[0], edge_index[1]]).astype(jnp.int32)
    dst_all = jnp.concatenate([edge_index[1], edge_index[0]]).astype(jnp.int32)
    t_all = jnp.concatenate([edge_time, edge_time]).astype(jnp.float32)
    pad = E_PAD - E_ALL
    src_p = jnp.pad(src_all, (0, pad), constant_values=-1)[None, :]
    dst_p = jnp.pad(dst_all, (0, pad), constant_values=0)[None, :]
    t_p = jnp.pad(t_all, (0, pad), constant_values=0.0)[None, :]

    n0 = jnp.broadcast_to(source_nodes.astype(jnp.int32)[:, None],
                          (BATCH, NUM_WALKS)).reshape(GROUPS, 8, 1)
    t0 = jnp.broadcast_to(current_times.astype(jnp.float32)[:, None],
                          (BATCH, NUM_WALKS)).reshape(GROUPS, 8, 1)

    full = pl.BlockSpec((1, E_PAD), lambda g: (0, 0))
    state = pl.BlockSpec((1, 8, 1), lambda g: (g, 0, 0))
    out3 = pl.BlockSpec((1, 8, WALK_LEN), lambda g: (g, 0, 0))

    on, ot, om = pl.pallas_call(
        _walks_kernel,
        grid=(GROUPS,),
        in_specs=[full, full, full, state, state],
        out_specs=[out3, out3, out3],
        out_shape=[
            jax.ShapeDtypeStruct((GROUPS, 8, WALK_LEN), jnp.int32),
            jax.ShapeDtypeStruct((GROUPS, 8, WALK_LEN), jnp.float32),
            jax.ShapeDtypeStruct((GROUPS, 8, WALK_LEN), jnp.float32),
        ],
    )(src_p, dst_p, t_p, n0, t0)

    walk_nodes = on.reshape(BATCH, NUM_WALKS, WALK_LEN)
    walk_times = ot.reshape(BATCH, NUM_WALKS, WALK_LEN)
    walk_masks = om.reshape(BATCH, NUM_WALKS, WALK_LEN)

    restart_probs = pl.pallas_call(
        _restart_kernel,
        in_specs=[
            pl.BlockSpec(memory_space=pltpu.SMEM),
            pl.BlockSpec((BATCH, 1), lambda: (0, 0)),
            pl.BlockSpec((NUM_NODES, MEMORY_DIM), lambda: (0, 0)),
            pl.BlockSpec((1, MEMORY_DIM + TIME_DIM), lambda: (0, 0)),
            pl.BlockSpec((1, 1), lambda: (0, 0)),
            pl.BlockSpec((1, TIME_DIM), lambda: (0, 0)),
            pl.BlockSpec((1, TIME_DIM), lambda: (0, 0)),
        ],
        out_specs=pl.BlockSpec((BATCH, 1), lambda: (0, 0)),
        out_shape=jax.ShapeDtypeStruct((BATCH, 1), jnp.float32),
        scratch_shapes=[pltpu.VMEM((BATCH, MEMORY_DIM), jnp.float32)],
    )(source_nodes.astype(jnp.int32),
      current_times.astype(jnp.float32)[:, None],
      memory_states.astype(jnp.float32),
      W_restart.astype(jnp.float32).reshape(1, -1),
      b_restart.astype(jnp.float32).reshape(1, 1),
      time_w.astype(jnp.float32)[None, :],
      time_b.astype(jnp.float32)[None, :])

    return walk_nodes, walk_times, walk_masks, restart_probs


# speedup vs baseline: 6.9473x; 1.8690x over previous
import numpy as np
import jax
import jax.numpy as jnp
from jax.experimental import pallas as pl
from jax.experimental.pallas import tpu as pltpu

NUM_NODES = 10000
NUM_EDGES = 50000
BATCH = 32
MEMORY_DIM = 128
TIME_DIM = 64
NUM_WALKS = 10
WALK_LEN = 3
TEMPERATURE = 0.1

E_ALL = 2 * NUM_EDGES
W_TOT = BATCH * NUM_WALKS
GROUPS = W_TOT // 8
CHUNK = 4096
N_CHUNKS = -(-E_ALL // CHUNK)
E_PAD = N_CHUNKS * CHUNK

_TINY = np.float32(np.finfo(np.float32).tiny)
_NEG_INF = np.float32(-np.inf)


def _np_threefry_pair(k0, k1, x0, x1):
    x0 = np.asarray(x0, np.uint32).copy()
    x1 = np.asarray(x1, np.uint32).copy()
    ks0 = np.uint32(k0)
    ks1 = np.uint32(k1)
    ks2 = np.uint32(ks0 ^ ks1 ^ np.uint32(0x1BD11BDA))
    rots = ((13, 15, 26, 6), (17, 29, 16, 24))
    sched = ((ks1, ks2), (ks2, ks0), (ks0, ks1), (ks1, ks2), (ks2, ks0))
    x0 = (x0 + ks0).astype(np.uint32)
    x1 = (x1 + ks1).astype(np.uint32)
    for i in range(5):
        for r in rots[i % 2]:
            x0 = (x0 + x1).astype(np.uint32)
            x1 = ((x1 << np.uint32(r)) | (x1 >> np.uint32(32 - r))).astype(np.uint32)
            x1 = (x1 ^ x0).astype(np.uint32)
        a, b = sched[i]
        x0 = (x0 + a).astype(np.uint32)
        x1 = (x1 + b + np.uint32(i + 1)).astype(np.uint32)
    return x0, x1


def _derive_step_keys():
    keys = []
    k = (np.uint32(0), np.uint32(1234))
    for _ in range(WALK_LEN - 1):
        h0, h1 = _np_threefry_pair(k[0], k[1], np.zeros(2, np.uint32),
                                   np.arange(2, dtype=np.uint32))
        k = (h0[0], h1[0])
        keys.append((int(h0[1]), int(h1[1])))
    return keys


_STEP_KEYS = _derive_step_keys()


def _tf_bits(x1, k0, k1):
    ks0 = jnp.uint32(k0)
    ks1 = jnp.uint32(k1)
    ks2 = jnp.uint32(k0 ^ k1 ^ 0x1BD11BDA)
    rots = ((13, 15, 26, 6), (17, 29, 16, 24))
    sched = ((ks1, ks2), (ks2, ks0), (ks0, ks1), (ks1, ks2), (ks2, ks0))
    x0 = jnp.full_like(x1, ks0)
    x1 = x1 + ks1
    for i in range(5):
        for r in rots[i % 2]:
            x0 = x0 + x1
            x1 = (x1 << jnp.uint32(r)) | (x1 >> jnp.uint32(32 - r))
            x1 = x1 ^ x0
        a, b = sched[i]
        x0 = x0 + a
        x1 = x1 + b + jnp.uint32(i + 1)
    return x0 ^ x1


def _gumbel_from_bits(bits):
    fb = (bits >> jnp.uint32(9)) | jnp.uint32(0x3F800000)
    f = jax.lax.bitcast_convert_type(fb, jnp.float32) - jnp.float32(1.0)
    u = jnp.maximum(_TINY, f * (jnp.float32(1.0) - _TINY) + _TINY)
    return -jnp.log(-jnp.log(u))


def _walks_kernel(src_ref, dst_ref, t_ref, n0_ref, t0_ref,
                  on_ref, ot_ref, om_ref):
    g = pl.program_id(0)
    cn = n0_ref[0]
    ct = t0_ref[0]
    alive = jnp.ones((8, 1), dtype=jnp.bool_)
    wk = jax.lax.broadcasted_iota(jnp.int32, (8, 1), 0) + 8 * g

    on_ref[0, :, 0:1] = cn
    ot_ref[0, :, 0:1] = ct
    om_ref[0, :, 0:1] = jnp.ones((8, 1), jnp.float32)

    for s in range(WALK_LEN - 1):
        k0, k1 = _STEP_KEYS[s]

        def p1_body(i, carry):
            bt_, bd_ = carry
            src = src_ref[:, pl.ds(i * CHUNK, CHUNK)]
            dst = dst_ref[:, pl.ds(i * CHUNK, CHUNK)]
            tt = t_ref[:, pl.ds(i * CHUNK, CHUNK)]
            valid = (src == cn) & (tt < ct)
            tc = jnp.where(valid, tt, _NEG_INF)
            cmax = jnp.max(tc, axis=1, keepdims=True)
            j = i * CHUNK + jax.lax.broadcasted_iota(jnp.int32, (1, CHUNK), 1)
            jbig = jnp.where(tc == cmax, j, jnp.int32(2**31 - 1))
            jsel = jnp.min(jbig, axis=1, keepdims=True)
            onehot = j == jsel
            dsel = jnp.sum(jnp.where(onehot, dst, 0), axis=1, keepdims=True)
            upd = cmax > bt_
            return (jnp.where(upd, cmax, bt_), jnp.where(upd, dsel, bd_))

        tmax_raw, bd1 = jax.lax.fori_loop(
            0, N_CHUNKS, p1_body,
            (jnp.full((8, 1), _NEG_INF, jnp.float32),
             jnp.zeros((8, 1), jnp.int32)))
        has_valid = tmax_raw > _NEG_INF
        tmax = jnp.where(has_valid, tmax_raw, jnp.float32(0.0))

        thr = tmax_raw - jnp.float32(3.0)

        def pb_body(i, cnt):
            src = src_ref[:, pl.ds(i * CHUNK, CHUNK)]
            tt = t_ref[:, pl.ds(i * CHUNK, CHUNK)]
            valid = (src == cn) & (tt < ct)
            m = valid & (tt >= thr)
            return cnt + jnp.sum(m.astype(jnp.int32), axis=1, keepdims=True)

        cnt = jax.lax.fori_loop(0, N_CHUNKS, pb_body,
                                jnp.zeros((8, 1), jnp.int32))
        multi = jnp.any(cnt >= 2)

        def p2_body(i, carry):
            bv, bd, bt = carry
            src = src_ref[:, pl.ds(i * CHUNK, CHUNK)]
            dst = dst_ref[:, pl.ds(i * CHUNK, CHUNK)]
            tt = t_ref[:, pl.ds(i * CHUNK, CHUNK)]
            valid = (src == cn) & (tt < ct)
            j = i * CHUNK + jax.lax.broadcasted_iota(jnp.int32, (1, CHUNK), 1)
            idxs = (wk * E_ALL + j).astype(jnp.uint32)
            gmb = _gumbel_from_bits(_tf_bits(idxs, k0, k1))
            logits = jnp.where(valid, (tt - tmax) / jnp.float32(TEMPERATURE),
                               _NEG_INF)
            score = logits + gmb
            cmax = jnp.max(score, axis=1, keepdims=True)
            jbig = jnp.where(score == cmax, j, jnp.int32(2**31 - 1))
            jsel = jnp.min(jbig, axis=1, keepdims=True)
            onehot = j == jsel
            dsel = jnp.sum(jnp.where(onehot, dst, 0), axis=1, keepdims=True)
            tsel = jnp.sum(jnp.where(onehot, tt, jnp.float32(0.0)),
                           axis=1, keepdims=True)
            upd = cmax > bv
            return (jnp.where(upd, cmax, bv),
                    jnp.where(upd, dsel, bd),
                    jnp.where(upd, tsel, bt))

        def full_path():
            _, bd2, bt2 = jax.lax.fori_loop(
                0, N_CHUNKS, p2_body,
                (jnp.full((8, 1), _NEG_INF, jnp.float32),
                 jnp.zeros((8, 1), jnp.int32),
                 jnp.zeros((8, 1), jnp.float32)))
            return bd2, bt2

        def fast_path():
            return bd1, tmax_raw

        bd, bt = jax.lax.cond(multi, full_path, fast_path)

        alive = alive & has_valid
        cn = jnp.where(alive, bd, cn)
        ct = jnp.where(alive, bt, ct)
        on_ref[0, :, s + 1:s + 2] = jnp.where(alive, bd, 0)
        ot_ref[0, :, s + 1:s + 2] = jnp.where(alive, bt, jnp.float32(0.0))
        om_ref[0, :, s + 1:s + 2] = alive.astype(jnp.float32)


def _restart_kernel(sn_ref, ct_ref, mem_ref, w_ref, b_ref, tw_ref, tb_ref,
                    out_ref, mrows_ref):
    def gather_body(i, _):
        idx = sn_ref[i]
        mrows_ref[pl.ds(i, 1), :] = mem_ref[pl.ds(idx, 1), :]
        return 0

    jax.lax.fori_loop(0, BATCH, gather_body, 0)
    mem = mrows_ref[...]
    te = jnp.cos(ct_ref[...] * tw_ref[...] + tb_ref[...])
    wm = w_ref[:, :MEMORY_DIM]
    wt = w_ref[:, MEMORY_DIM:]
    r = (jnp.sum(mem * wm, axis=1, keepdims=True)
         + jnp.sum(te * wt, axis=1, keepdims=True) + b_ref[...])
    out_ref[...] = jax.nn.sigmoid(r)


def kernel(source_nodes, current_times, edge_index, edge_time, memory_states,
           W_restart, b_restart, time_w, time_b):
    src_all = jnp.concatenate([edge_index[0], edge_index[1]]).astype(jnp.int32)
    dst_all = jnp.concatenate([edge_index[1], edge_index[0]]).astype(jnp.int32)
    t_all = jnp.concatenate([edge_time, edge_time]).astype(jnp.float32)
    pad = E_PAD - E_ALL
    src_p = jnp.pad(src_all, (0, pad), constant_values=-1)[None, :]
    dst_p = jnp.pad(dst_all, (0, pad), constant_values=0)[None, :]
    t_p = jnp.pad(t_all, (0, pad), constant_values=0.0)[None, :]

    n0 = jnp.broadcast_to(source_nodes.astype(jnp.int32)[:, None],
                          (BATCH, NUM_WALKS)).reshape(GROUPS, 8, 1)
    t0 = jnp.broadcast_to(current_times.astype(jnp.float32)[:, None],
                          (BATCH, NUM_WALKS)).reshape(GROUPS, 8, 1)

    full = pl.BlockSpec((1, E_PAD), lambda g: (0, 0))
    state = pl.BlockSpec((1, 8, 1), lambda g: (g, 0, 0))
    out3 = pl.BlockSpec((1, 8, WALK_LEN), lambda g: (g, 0, 0))

    on, ot, om = pl.pallas_call(
        _walks_kernel,
        grid=(GROUPS,),
        in_specs=[full, full, full, state, state],
        out_specs=[out3, out3, out3],
        out_shape=[
            jax.ShapeDtypeStruct((GROUPS, 8, WALK_LEN), jnp.int32),
            jax.ShapeDtypeStruct((GROUPS, 8, WALK_LEN), jnp.float32),
            jax.ShapeDtypeStruct((GROUPS, 8, WALK_LEN), jnp.float32),
        ],
    )(src_p, dst_p, t_p, n0, t0)

    walk_nodes = on.reshape(BATCH, NUM_WALKS, WALK_LEN)
    walk_times = ot.reshape(BATCH, NUM_WALKS, WALK_LEN)
    walk_masks = om.reshape(BATCH, NUM_WALKS, WALK_LEN)

    restart_probs = pl.pallas_call(
        _restart_kernel,
        in_specs=[
            pl.BlockSpec(memory_space=pltpu.SMEM),
            pl.BlockSpec((BATCH, 1), lambda: (0, 0)),
            pl.BlockSpec((NUM_NODES, MEMORY_DIM), lambda: (0, 0)),
            pl.BlockSpec((1, MEMORY_DIM + TIME_DIM), lambda: (0, 0)),
            pl.BlockSpec((1, 1), lambda: (0, 0)),
            pl.BlockSpec((1, TIME_DIM), lambda: (0, 0)),
            pl.BlockSpec((1, TIME_DIM), lambda: (0, 0)),
        ],
        out_specs=pl.BlockSpec((BATCH, 1), lambda: (0, 0)),
        out_shape=jax.ShapeDtypeStruct((BATCH, 1), jnp.float32),
        scratch_shapes=[pltpu.VMEM((BATCH, MEMORY_DIM), jnp.float32)],
    )(source_nodes.astype(jnp.int32),
      current_times.astype(jnp.float32)[:, None],
      memory_states.astype(jnp.float32),
      W_restart.astype(jnp.float32).reshape(1, -1),
      b_restart.astype(jnp.float32).reshape(1, 1),
      time_w.astype(jnp.float32)[None, :],
      time_b.astype(jnp.float32)[None, :])

    return walk_nodes, walk_times, walk_masks, restart_probs


# speedup vs baseline: 13.1520x; 1.8931x over previous
import numpy as np
import jax
import jax.numpy as jnp
from jax.experimental import pallas as pl
from jax.experimental.pallas import tpu as pltpu

NUM_NODES = 10000
NUM_EDGES = 50000
BATCH = 32
MEMORY_DIM = 128
TIME_DIM = 64
NUM_WALKS = 10
WALK_LEN = 3
TEMPERATURE = 0.1

E_ALL = 2 * NUM_EDGES
W_TOT = BATCH * NUM_WALKS
GROUPS = W_TOT // 8
C1 = 8192
C2 = 1024
N1 = 13
N2 = 104
E_PAD = N1 * C1

_TINY = np.float32(np.finfo(np.float32).tiny)
_NEG_INF = np.float32(-np.inf)


def _np_threefry_pair(k0, k1, x0, x1):
    x0 = np.asarray(x0, np.uint32).copy()
    x1 = np.asarray(x1, np.uint32).copy()
    ks0 = np.uint32(k0)
    ks1 = np.uint32(k1)
    ks2 = np.uint32(ks0 ^ ks1 ^ np.uint32(0x1BD11BDA))
    rots = ((13, 15, 26, 6), (17, 29, 16, 24))
    sched = ((ks1, ks2), (ks2, ks0), (ks0, ks1), (ks1, ks2), (ks2, ks0))
    x0 = (x0 + ks0).astype(np.uint32)
    x1 = (x1 + ks1).astype(np.uint32)
    for i in range(5):
        for r in rots[i % 2]:
            x0 = (x0 + x1).astype(np.uint32)
            x1 = ((x1 << np.uint32(r)) | (x1 >> np.uint32(32 - r))).astype(np.uint32)
            x1 = (x1 ^ x0).astype(np.uint32)
        a, b = sched[i]
        x0 = (x0 + a).astype(np.uint32)
        x1 = (x1 + b + np.uint32(i + 1)).astype(np.uint32)
    return x0, x1


def _derive_step_keys():
    keys = []
    k = (np.uint32(0), np.uint32(1234))
    for _ in range(WALK_LEN - 1):
        h0, h1 = _np_threefry_pair(k[0], k[1], np.zeros(2, np.uint32),
                                   np.arange(2, dtype=np.uint32))
        k = (h0[0], h1[0])
        keys.append((int(h0[1]), int(h1[1])))
    return keys


_STEP_KEYS = _derive_step_keys()


def _tf_bits(x1, k0, k1):
    ks0 = jnp.uint32(k0)
    ks1 = jnp.uint32(k1)
    ks2 = jnp.uint32(k0 ^ k1 ^ 0x1BD11BDA)
    rots = ((13, 15, 26, 6), (17, 29, 16, 24))
    sched = ((ks1, ks2), (ks2, ks0), (ks0, ks1), (ks1, ks2), (ks2, ks0))
    x0 = jnp.full_like(x1, ks0)
    x1 = x1 + ks1
    for i in range(5):
        for r in rots[i % 2]:
            x0 = x0 + x1
            x1 = (x1 << jnp.uint32(r)) | (x1 >> jnp.uint32(32 - r))
            x1 = x1 ^ x0
        a, b = sched[i]
        x0 = x0 + a
        x1 = x1 + b + jnp.uint32(i + 1)
    return x0 ^ x1


def _gumbel_from_bits(bits):
    fb = (bits >> jnp.uint32(9)) | jnp.uint32(0x3F800000)
    f = jax.lax.bitcast_convert_type(fb, jnp.float32) - jnp.float32(1.0)
    u = jnp.maximum(_TINY, f * (jnp.float32(1.0) - _TINY) + _TINY)
    return -jnp.log(-jnp.log(u))


def _walks_kernel(src_ref, dst_ref, t_ref, n0_ref, t0_ref,
                  on_ref, ot_ref, om_ref):
    g = pl.program_id(0)
    cn = n0_ref[0]
    ct = t0_ref[0]
    alive = jnp.ones((8, 1), dtype=jnp.bool_)
    wk = jax.lax.broadcasted_iota(jnp.int32, (8, 1), 0) + 8 * g

    on_ref[0, :, 0:1] = cn
    ot_ref[0, :, 0:1] = ct
    om_ref[0, :, 0:1] = jnp.ones((8, 1), jnp.float32)

    for s in range(WALK_LEN - 1):
        k0, k1 = _STEP_KEYS[s]

        def scan_top2(cnx, ctx):
            rows = cnx.shape[0]

            def p1_body(i, carry):
                bt1_, bt2_, bd_ = carry
                src = src_ref[:, pl.ds(i * C1, C1)]
                dst = dst_ref[:, pl.ds(i * C1, C1)]
                tt = t_ref[:, pl.ds(i * C1, C1)]
                valid = (src == cnx) & (tt < ctx)
                tc = jnp.where(valid, tt, _NEG_INF)
                cmax = jnp.max(tc, axis=1, keepdims=True)
                j = jax.lax.broadcasted_iota(jnp.int32, (1, C1), 1)
                jbig = jnp.where(tc == cmax, j, jnp.int32(2**31 - 1))
                jsel = jnp.min(jbig, axis=1, keepdims=True)
                onehot = j == jsel
                dsel = jnp.sum(jnp.where(onehot, dst, 0),
                               axis=1, keepdims=True)
                cmax2 = jnp.max(jnp.where(onehot, _NEG_INF, tc),
                                axis=1, keepdims=True)
                nb1 = jnp.maximum(bt1_, cmax)
                nb2 = jnp.maximum(jnp.minimum(bt1_, cmax),
                                  jnp.maximum(bt2_, cmax2))
                nbd = jnp.where(cmax > bt1_, dsel, bd_)
                return nb1, nb2, nbd

            return jax.lax.fori_loop(
                0, N1, p1_body,
                (jnp.full((rows, 1), _NEG_INF, jnp.float32),
                 jnp.full((rows, 1), _NEG_INF, jnp.float32),
                 jnp.zeros((rows, 1), jnp.int32)))

        if s == 0:
            cn2 = jnp.concatenate([cn[0:1], cn[7:8]], axis=0)
            ct2 = jnp.concatenate([ct[0:1], ct[7:8]], axis=0)
            b1_2, b2_2, bd_2 = scan_top2(cn2, ct2)
            sel = (wk // 10) == ((8 * g) // 10)
            tmax_raw = jnp.where(sel, b1_2[0:1], b1_2[1:2])
            t2_raw = jnp.where(sel, b2_2[0:1], b2_2[1:2])
            bd1 = jnp.where(sel, bd_2[0:1], bd_2[1:2])
        else:
            tmax_raw, t2_raw, bd1 = scan_top2(cn, ct)
        has_valid = tmax_raw > _NEG_INF
        tmax = jnp.where(has_valid, tmax_raw, jnp.float32(0.0))
        multi = jnp.any((t2_raw >= tmax_raw - jnp.float32(3.0))
                        & (t2_raw > _NEG_INF))

        def p2_body(i, carry):
            bv, bd, bt = carry
            src = src_ref[:, pl.ds(i * C2, C2)]
            dst = dst_ref[:, pl.ds(i * C2, C2)]
            tt = t_ref[:, pl.ds(i * C2, C2)]
            valid = (src == cn) & (tt < ct)
            j = i * C2 + jax.lax.broadcasted_iota(jnp.int32, (1, C2), 1)
            idxs = (wk * E_ALL + j).astype(jnp.uint32)
            gmb = _gumbel_from_bits(_tf_bits(idxs, k0, k1))
            logits = jnp.where(valid, (tt - tmax) / jnp.float32(TEMPERATURE),
                               _NEG_INF)
            score = logits + gmb
            cmax = jnp.max(score, axis=1, keepdims=True)
            jbig = jnp.where(score == cmax, j, jnp.int32(2**31 - 1))
            jsel = jnp.min(jbig, axis=1, keepdims=True)
            onehot = j == jsel
            dsel = jnp.sum(jnp.where(onehot, dst, 0), axis=1, keepdims=True)
            tsel = jnp.sum(jnp.where(onehot, tt, jnp.float32(0.0)),
                           axis=1, keepdims=True)
            upd = cmax > bv
            return (jnp.where(upd, cmax, bv),
                    jnp.where(upd, dsel, bd),
                    jnp.where(upd, tsel, bt))

        def full_path():
            _, bd2, bt2 = jax.lax.fori_loop(
                0, N2, p2_body,
                (jnp.full((8, 1), _NEG_INF, jnp.float32),
                 jnp.zeros((8, 1), jnp.int32),
                 jnp.zeros((8, 1), jnp.float32)))
            return bd2, bt2

        def fast_path():
            return bd1, tmax_raw

        bd, bt = jax.lax.cond(multi, full_path, fast_path)

        alive = alive & has_valid
        cn = jnp.where(alive, bd, cn)
        ct = jnp.where(alive, bt, ct)
        on_ref[0, :, s + 1:s + 2] = jnp.where(alive, bd, 0)
        ot_ref[0, :, s + 1:s + 2] = jnp.where(alive, bt, jnp.float32(0.0))
        om_ref[0, :, s + 1:s + 2] = alive.astype(jnp.float32)


def _restart_kernel(sn_ref, ct_ref, mem_ref, w_ref, b_ref, tw_ref, tb_ref,
                    out_ref, mrows_ref):
    def gather_body(i, _):
        idx = sn_ref[i]
        mrows_ref[pl.ds(i, 1), :] = mem_ref[pl.ds(idx, 1), :]
        return 0

    jax.lax.fori_loop(0, BATCH, gather_body, 0)
    mem = mrows_ref[...]
    te = jnp.cos(ct_ref[...] * tw_ref[...] + tb_ref[...])
    wm = w_ref[:, :MEMORY_DIM]
    wt = w_ref[:, MEMORY_DIM:]
    r = (jnp.sum(mem * wm, axis=1, keepdims=True)
         + jnp.sum(te * wt, axis=1, keepdims=True) + b_ref[...])
    out_ref[...] = jax.nn.sigmoid(r)


def kernel(source_nodes, current_times, edge_index, edge_time, memory_states,
           W_restart, b_restart, time_w, time_b):
    src_all = jnp.concatenate([edge_index[0], edge_index[1]]).astype(jnp.int32)
    dst_all = jnp.concatenate([edge_index[1], edge_index[0]]).astype(jnp.int32)
    t_all = jnp.concatenate([edge_time, edge_time]).astype(jnp.float32)
    pad = E_PAD - E_ALL
    src_p = jnp.pad(src_all, (0, pad), constant_values=-1)[None, :]
    dst_p = jnp.pad(dst_all, (0, pad), constant_values=0)[None, :]
    t_p = jnp.pad(t_all, (0, pad), constant_values=0.0)[None, :]

    n0 = jnp.broadcast_to(source_nodes.astype(jnp.int32)[:, None],
                          (BATCH, NUM_WALKS)).reshape(GROUPS, 8, 1)
    t0 = jnp.broadcast_to(current_times.astype(jnp.float32)[:, None],
                          (BATCH, NUM_WALKS)).reshape(GROUPS, 8, 1)

    full = pl.BlockSpec((1, E_PAD), lambda g: (0, 0))
    state = pl.BlockSpec((1, 8, 1), lambda g: (g, 0, 0))
    out3 = pl.BlockSpec((1, 8, WALK_LEN), lambda g: (g, 0, 0))

    on, ot, om = pl.pallas_call(
        _walks_kernel,
        grid=(GROUPS,),
        compiler_params=pltpu.CompilerParams(
            dimension_semantics=("parallel",)),
        in_specs=[full, full, full, state, state],
        out_specs=[out3, out3, out3],
        out_shape=[
            jax.ShapeDtypeStruct((GROUPS, 8, WALK_LEN), jnp.int32),
            jax.ShapeDtypeStruct((GROUPS, 8, WALK_LEN), jnp.float32),
            jax.ShapeDtypeStruct((GROUPS, 8, WALK_LEN), jnp.float32),
        ],
    )(src_p, dst_p, t_p, n0, t0)

    walk_nodes = on.reshape(BATCH, NUM_WALKS, WALK_LEN)
    walk_times = ot.reshape(BATCH, NUM_WALKS, WALK_LEN)
    walk_masks = om.reshape(BATCH, NUM_WALKS, WALK_LEN)

    restart_probs = pl.pallas_call(
        _restart_kernel,
        in_specs=[
            pl.BlockSpec(memory_space=pltpu.SMEM),
            pl.BlockSpec((BATCH, 1), lambda: (0, 0)),
            pl.BlockSpec((NUM_NODES, MEMORY_DIM), lambda: (0, 0)),
            pl.BlockSpec((1, MEMORY_DIM + TIME_DIM), lambda: (0, 0)),
            pl.BlockSpec((1, 1), lambda: (0, 0)),
            pl.BlockSpec((1, TIME_DIM), lambda: (0, 0)),
            pl.BlockSpec((1, TIME_DIM), lambda: (0, 0)),
        ],
        out_specs=pl.BlockSpec((BATCH, 1), lambda: (0, 0)),
        out_shape=jax.ShapeDtypeStruct((BATCH, 1), jnp.float32),
        scratch_shapes=[pltpu.VMEM((BATCH, MEMORY_DIM), jnp.float32)],
    )(source_nodes.astype(jnp.int32),
      current_times.astype(jnp.float32)[:, None],
      memory_states.astype(jnp.float32),
      W_restart.astype(jnp.float32).reshape(1, -1),
      b_restart.astype(jnp.float32).reshape(1, 1),
      time_w.astype(jnp.float32)[None, :],
      time_b.astype(jnp.float32)[None, :])

    return walk_nodes, walk_times, walk_masks, restart_probs


# speedup vs baseline: 18.8220x; 1.4311x over previous
import numpy as np
import jax
import jax.numpy as jnp
from jax.experimental import pallas as pl
from jax.experimental.pallas import tpu as pltpu

NUM_NODES = 10000
NUM_EDGES = 50000
BATCH = 32
MEMORY_DIM = 128
TIME_DIM = 64
NUM_WALKS = 10
WALK_LEN = 3
TEMPERATURE = 0.1

E_ALL = 2 * NUM_EDGES
W_TOT = BATCH * NUM_WALKS
GROUPS = W_TOT // 8
C1 = 6272
C2 = 1024
N1P = 8
N2 = 98
E_PAD = 2 * N1P * C1

_TINY = np.float32(np.finfo(np.float32).tiny)
_NEG_INF = np.float32(-np.inf)


def _np_threefry_pair(k0, k1, x0, x1):
    x0 = np.asarray(x0, np.uint32).copy()
    x1 = np.asarray(x1, np.uint32).copy()
    ks0 = np.uint32(k0)
    ks1 = np.uint32(k1)
    ks2 = np.uint32(ks0 ^ ks1 ^ np.uint32(0x1BD11BDA))
    rots = ((13, 15, 26, 6), (17, 29, 16, 24))
    sched = ((ks1, ks2), (ks2, ks0), (ks0, ks1), (ks1, ks2), (ks2, ks0))
    x0 = (x0 + ks0).astype(np.uint32)
    x1 = (x1 + ks1).astype(np.uint32)
    for i in range(5):
        for r in rots[i % 2]:
            x0 = (x0 + x1).astype(np.uint32)
            x1 = ((x1 << np.uint32(r)) | (x1 >> np.uint32(32 - r))).astype(np.uint32)
            x1 = (x1 ^ x0).astype(np.uint32)
        a, b = sched[i]
        x0 = (x0 + a).astype(np.uint32)
        x1 = (x1 + b + np.uint32(i + 1)).astype(np.uint32)
    return x0, x1


def _derive_step_keys():
    keys = []
    k = (np.uint32(0), np.uint32(1234))
    for _ in range(WALK_LEN - 1):
        h0, h1 = _np_threefry_pair(k[0], k[1], np.zeros(2, np.uint32),
                                   np.arange(2, dtype=np.uint32))
        k = (h0[0], h1[0])
        keys.append((int(h0[1]), int(h1[1])))
    return keys


_STEP_KEYS = _derive_step_keys()


def _tf_bits(x1, k0, k1):
    ks0 = jnp.uint32(k0)
    ks1 = jnp.uint32(k1)
    ks2 = jnp.uint32(k0 ^ k1 ^ 0x1BD11BDA)
    rots = ((13, 15, 26, 6), (17, 29, 16, 24))
    sched = ((ks1, ks2), (ks2, ks0), (ks0, ks1), (ks1, ks2), (ks2, ks0))
    x0 = jnp.full_like(x1, ks0)
    x1 = x1 + ks1
    for i in range(5):
        for r in rots[i % 2]:
            x0 = x0 + x1
            x1 = (x1 << jnp.uint32(r)) | (x1 >> jnp.uint32(32 - r))
            x1 = x1 ^ x0
        a, b = sched[i]
        x0 = x0 + a
        x1 = x1 + b + jnp.uint32(i + 1)
    return x0 ^ x1


def _gumbel_from_bits(bits):
    fb = (bits >> jnp.uint32(9)) | jnp.uint32(0x3F800000)
    f = jax.lax.bitcast_convert_type(fb, jnp.float32) - jnp.float32(1.0)
    u = jnp.maximum(_TINY, f * (jnp.float32(1.0) - _TINY) + _TINY)
    return -jnp.log(-jnp.log(u))


def _walks_kernel(src_ref, dst_ref, t_ref, n0_ref, t0_ref,
                  on_ref, ot_ref, om_ref):
    g = pl.program_id(0)
    cn = n0_ref[0]
    ct = t0_ref[0]
    alive = jnp.ones((8, 1), dtype=jnp.bool_)
    wk = jax.lax.broadcasted_iota(jnp.int32, (8, 1), 0) + 8 * g

    on_ref[0, :, 0:1] = cn
    ot_ref[0, :, 0:1] = ct
    om_ref[0, :, 0:1] = jnp.ones((8, 1), jnp.float32)

    for s in range(WALK_LEN - 1):
        k0, k1 = _STEP_KEYS[s]

        def scan_top2(cnx, ctx):
            rows = cnx.shape[0]

            def chunk_upd(off, bt1_, bt2_, bd_):
                src = src_ref[:, pl.ds(off, C1)]
                dst = dst_ref[:, pl.ds(off, C1)]
                tt = t_ref[:, pl.ds(off, C1)]
                valid = (src == cnx) & (tt < ctx)
                tc = jnp.where(valid, tt, _NEG_INF)
                cmax = jnp.max(tc, axis=1, keepdims=True)
                j = jax.lax.broadcasted_iota(jnp.int32, (1, C1), 1)
                jbig = jnp.where(tc == cmax, j, jnp.int32(2**31 - 1))
                jsel = jnp.min(jbig, axis=1, keepdims=True)
                onehot = j == jsel
                dsel = jnp.sum(jnp.where(onehot, dst, 0),
                               axis=1, keepdims=True)
                cmax2 = jnp.max(jnp.where(onehot, _NEG_INF, tc),
                                axis=1, keepdims=True)
                nb1 = jnp.maximum(bt1_, cmax)
                nb2 = jnp.maximum(jnp.minimum(bt1_, cmax),
                                  jnp.maximum(bt2_, cmax2))
                nbd = jnp.where(cmax > bt1_, dsel, bd_)
                return nb1, nb2, nbd

            def p1_body(i, carry):
                a1, a2, ad, b1, b2, bd_ = carry
                a1, a2, ad = chunk_upd(2 * i * C1, a1, a2, ad)
                b1, b2, bd_ = chunk_upd((2 * i + 1) * C1, b1, b2, bd_)
                return a1, a2, ad, b1, b2, bd_

            neg = jnp.full((rows, 1), _NEG_INF, jnp.float32)
            zero = jnp.zeros((rows, 1), jnp.int32)
            a1, a2, ad, b1, b2, bd_ = jax.lax.fori_loop(
                0, N1P, p1_body, (neg, neg, zero, neg, neg, zero))
            m1 = jnp.maximum(a1, b1)
            m2 = jnp.maximum(jnp.minimum(a1, b1), jnp.maximum(a2, b2))
            md = jnp.where(a1 >= b1, ad, bd_)
            return m1, m2, md

        if s == 0:
            cn2 = jnp.concatenate([cn[0:1], cn[7:8]], axis=0)
            ct2 = jnp.concatenate([ct[0:1], ct[7:8]], axis=0)
            b1_2, b2_2, bd_2 = scan_top2(cn2, ct2)
            sel = (wk // 10) == ((8 * g) // 10)
            tmax_raw = jnp.where(sel, b1_2[0:1], b1_2[1:2])
            t2_raw = jnp.where(sel, b2_2[0:1], b2_2[1:2])
            bd1 = jnp.where(sel, bd_2[0:1], bd_2[1:2])
        else:
            tmax_raw, t2_raw, bd1 = scan_top2(cn, ct)
        has_valid = tmax_raw > _NEG_INF
        tmax = jnp.where(has_valid, tmax_raw, jnp.float32(0.0))
        multi = jnp.any((t2_raw >= tmax_raw - jnp.float32(3.0))
                        & (t2_raw > _NEG_INF))

        def p2_body(i, carry):
            bv, bd, bt = carry
            src = src_ref[:, pl.ds(i * C2, C2)]
            dst = dst_ref[:, pl.ds(i * C2, C2)]
            tt = t_ref[:, pl.ds(i * C2, C2)]
            valid = (src == cn) & (tt < ct)
            j = i * C2 + jax.lax.broadcasted_iota(jnp.int32, (1, C2), 1)
            idxs = (wk * E_ALL + j).astype(jnp.uint32)
            gmb = _gumbel_from_bits(_tf_bits(idxs, k0, k1))
            logits = jnp.where(valid, (tt - tmax) / jnp.float32(TEMPERATURE),
                               _NEG_INF)
            score = logits + gmb
            cmax = jnp.max(score, axis=1, keepdims=True)
            jbig = jnp.where(score == cmax, j, jnp.int32(2**31 - 1))
            jsel = jnp.min(jbig, axis=1, keepdims=True)
            onehot = j == jsel
            dsel = jnp.sum(jnp.where(onehot, dst, 0), axis=1, keepdims=True)
            tsel = jnp.sum(jnp.where(onehot, tt, jnp.float32(0.0)),
                           axis=1, keepdims=True)
            upd = cmax > bv
            return (jnp.where(upd, cmax, bv),
                    jnp.where(upd, dsel, bd),
                    jnp.where(upd, tsel, bt))

        def full_path():
            _, bd2, bt2 = jax.lax.fori_loop(
                0, N2, p2_body,
                (jnp.full((8, 1), _NEG_INF, jnp.float32),
                 jnp.zeros((8, 1), jnp.int32),
                 jnp.zeros((8, 1), jnp.float32)))
            return bd2, bt2

        def fast_path():
            return bd1, tmax_raw

        bd, bt = jax.lax.cond(multi, full_path, fast_path)

        alive = alive & has_valid
        cn = jnp.where(alive, bd, cn)
        ct = jnp.where(alive, bt, ct)
        on_ref[0, :, s + 1:s + 2] = jnp.where(alive, bd, 0)
        ot_ref[0, :, s + 1:s + 2] = jnp.where(alive, bt, jnp.float32(0.0))
        om_ref[0, :, s + 1:s + 2] = alive.astype(jnp.float32)


def _restart_kernel(sn_ref, ct_ref, mem_ref, w_ref, b_ref, tw_ref, tb_ref,
                    out_ref, mrows_ref):
    def gather_body(i, _):
        idx = sn_ref[i]
        mrows_ref[pl.ds(i, 1), :] = mem_ref[pl.ds(idx, 1), :]
        return 0

    jax.lax.fori_loop(0, BATCH, gather_body, 0)
    mem = mrows_ref[...]
    te = jnp.cos(ct_ref[...] * tw_ref[...] + tb_ref[...])
    wm = w_ref[:, :MEMORY_DIM]
    wt = w_ref[:, MEMORY_DIM:]
    r = (jnp.sum(mem * wm, axis=1, keepdims=True)
         + jnp.sum(te * wt, axis=1, keepdims=True) + b_ref[...])
    out_ref[...] = jax.nn.sigmoid(r)


def kernel(source_nodes, current_times, edge_index, edge_time, memory_states,
           W_restart, b_restart, time_w, time_b):
    src_all = jnp.concatenate([edge_index[0], edge_index[1]]).astype(jnp.int32)
    dst_all = jnp.concatenate([edge_index[1], edge_index[0]]).astype(jnp.int32)
    t_all = jnp.concatenate([edge_time, edge_time]).astype(jnp.float32)
    pad = E_PAD - E_ALL
    src_p = jnp.pad(src_all, (0, pad), constant_values=-1)[None, :]
    dst_p = jnp.pad(dst_all, (0, pad), constant_values=0)[None, :]
    t_p = jnp.pad(t_all, (0, pad), constant_values=0.0)[None, :]

    n0 = jnp.broadcast_to(source_nodes.astype(jnp.int32)[:, None],
                          (BATCH, NUM_WALKS)).reshape(GROUPS, 8, 1)
    t0 = jnp.broadcast_to(current_times.astype(jnp.float32)[:, None],
                          (BATCH, NUM_WALKS)).reshape(GROUPS, 8, 1)

    full = pl.BlockSpec((1, E_PAD), lambda g: (0, 0))
    state = pl.BlockSpec((1, 8, 1), lambda g: (g, 0, 0))
    out3 = pl.BlockSpec((1, 8, WALK_LEN), lambda g: (g, 0, 0))

    on, ot, om = pl.pallas_call(
        _walks_kernel,
        grid=(GROUPS,),
        in_specs=[full, full, full, state, state],
        out_specs=[out3, out3, out3],
        out_shape=[
            jax.ShapeDtypeStruct((GROUPS, 8, WALK_LEN), jnp.int32),
            jax.ShapeDtypeStruct((GROUPS, 8, WALK_LEN), jnp.float32),
            jax.ShapeDtypeStruct((GROUPS, 8, WALK_LEN), jnp.float32),
        ],
    )(src_p, dst_p, t_p, n0, t0)

    walk_nodes = on.reshape(BATCH, NUM_WALKS, WALK_LEN)
    walk_times = ot.reshape(BATCH, NUM_WALKS, WALK_LEN)
    walk_masks = om.reshape(BATCH, NUM_WALKS, WALK_LEN)

    restart_probs = pl.pallas_call(
        _restart_kernel,
        in_specs=[
            pl.BlockSpec(memory_space=pltpu.SMEM),
            pl.BlockSpec((BATCH, 1), lambda: (0, 0)),
            pl.BlockSpec((NUM_NODES, MEMORY_DIM), lambda: (0, 0)),
            pl.BlockSpec((1, MEMORY_DIM + TIME_DIM), lambda: (0, 0)),
            pl.BlockSpec((1, 1), lambda: (0, 0)),
            pl.BlockSpec((1, TIME_DIM), lambda: (0, 0)),
            pl.BlockSpec((1, TIME_DIM), lambda: (0, 0)),
        ],
        out_specs=pl.BlockSpec((BATCH, 1), lambda: (0, 0)),
        out_shape=jax.ShapeDtypeStruct((BATCH, 1), jnp.float32),
        scratch_shapes=[pltpu.VMEM((BATCH, MEMORY_DIM), jnp.float32)],
    )(source_nodes.astype(jnp.int32),
      current_times.astype(jnp.float32)[:, None],
      memory_states.astype(jnp.float32),
      W_restart.astype(jnp.float32).reshape(1, -1),
      b_restart.astype(jnp.float32).reshape(1, 1),
      time_w.astype(jnp.float32)[None, :],
      time_b.astype(jnp.float32)[None, :])

    return walk_nodes, walk_times, walk_masks, restart_probs


# speedup vs baseline: 25.0462x; 1.3307x over previous
import numpy as np
import jax
import jax.numpy as jnp
from jax.experimental import pallas as pl
from jax.experimental.pallas import tpu as pltpu

NUM_NODES = 10000
NUM_EDGES = 50000
BATCH = 32
MEMORY_DIM = 128
TIME_DIM = 64
NUM_WALKS = 10
WALK_LEN = 3
TEMPERATURE = 0.1

E_ALL = 2 * NUM_EDGES
W_TOT = BATCH * NUM_WALKS
GROUPS = W_TOT // 8
C1 = 6272
C2 = 1024
N1P = 4
N2 = 98
E_PAD = 4 * N1P * C1

_TINY = np.float32(np.finfo(np.float32).tiny)
_NEG_INF = np.float32(-np.inf)


def _np_threefry_pair(k0, k1, x0, x1):
    x0 = np.asarray(x0, np.uint32).copy()
    x1 = np.asarray(x1, np.uint32).copy()
    ks0 = np.uint32(k0)
    ks1 = np.uint32(k1)
    ks2 = np.uint32(ks0 ^ ks1 ^ np.uint32(0x1BD11BDA))
    rots = ((13, 15, 26, 6), (17, 29, 16, 24))
    sched = ((ks1, ks2), (ks2, ks0), (ks0, ks1), (ks1, ks2), (ks2, ks0))
    x0 = (x0 + ks0).astype(np.uint32)
    x1 = (x1 + ks1).astype(np.uint32)
    for i in range(5):
        for r in rots[i % 2]:
            x0 = (x0 + x1).astype(np.uint32)
            x1 = ((x1 << np.uint32(r)) | (x1 >> np.uint32(32 - r))).astype(np.uint32)
            x1 = (x1 ^ x0).astype(np.uint32)
        a, b = sched[i]
        x0 = (x0 + a).astype(np.uint32)
        x1 = (x1 + b + np.uint32(i + 1)).astype(np.uint32)
    return x0, x1


def _derive_step_keys():
    keys = []
    k = (np.uint32(0), np.uint32(1234))
    for _ in range(WALK_LEN - 1):
        h0, h1 = _np_threefry_pair(k[0], k[1], np.zeros(2, np.uint32),
                                   np.arange(2, dtype=np.uint32))
        k = (h0[0], h1[0])
        keys.append((int(h0[1]), int(h1[1])))
    return keys


_STEP_KEYS = _derive_step_keys()


def _tf_bits(x1, k0, k1):
    ks0 = jnp.uint32(k0)
    ks1 = jnp.uint32(k1)
    ks2 = jnp.uint32(k0 ^ k1 ^ 0x1BD11BDA)
    rots = ((13, 15, 26, 6), (17, 29, 16, 24))
    sched = ((ks1, ks2), (ks2, ks0), (ks0, ks1), (ks1, ks2), (ks2, ks0))
    x0 = jnp.full_like(x1, ks0)
    x1 = x1 + ks1
    for i in range(5):
        for r in rots[i % 2]:
            x0 = x0 + x1
            x1 = (x1 << jnp.uint32(r)) | (x1 >> jnp.uint32(32 - r))
            x1 = x1 ^ x0
        a, b = sched[i]
        x0 = x0 + a
        x1 = x1 + b + jnp.uint32(i + 1)
    return x0 ^ x1


def _gumbel_from_bits(bits):
    fb = (bits >> jnp.uint32(9)) | jnp.uint32(0x3F800000)
    f = jax.lax.bitcast_convert_type(fb, jnp.float32) - jnp.float32(1.0)
    u = jnp.maximum(_TINY, f * (jnp.float32(1.0) - _TINY) + _TINY)
    return -jnp.log(-jnp.log(u))


def _walks_kernel(src_ref, dst_ref, t_ref, n0_ref, t0_ref,
                  on_ref, ot_ref, om_ref):
    g = pl.program_id(0)
    cn = n0_ref[0]
    ct = t0_ref[0]
    alive = jnp.ones((8, 1), dtype=jnp.bool_)
    wk = jax.lax.broadcasted_iota(jnp.int32, (8, 1), 0) + 8 * g

    on_ref[0, :, 0:1] = cn
    ot_ref[0, :, 0:1] = ct
    om_ref[0, :, 0:1] = jnp.ones((8, 1), jnp.float32)

    for s in range(WALK_LEN - 1):
        k0, k1 = _STEP_KEYS[s]

        def scan_top2(cnx, ctx):
            rows = cnx.shape[0]

            def chunk_upd(off, bt1_, bt2_, bd_):
                src = src_ref[:, pl.ds(off, C1)]
                dst = dst_ref[:, pl.ds(off, C1)]
                tt = t_ref[:, pl.ds(off, C1)]
                valid = (src == cnx) & (tt < ctx)
                tc = jnp.where(valid, tt, _NEG_INF)
                cmax = jnp.max(tc, axis=1, keepdims=True)
                j = jax.lax.broadcasted_iota(jnp.int32, (1, C1), 1)
                jbig = jnp.where(tc == cmax, j, jnp.int32(2**31 - 1))
                jsel = jnp.min(jbig, axis=1, keepdims=True)
                onehot = j == jsel
                dsel = jnp.sum(jnp.where(onehot, dst, 0),
                               axis=1, keepdims=True)
                cmax2 = jnp.max(jnp.where(onehot, _NEG_INF, tc),
                                axis=1, keepdims=True)
                nb1 = jnp.maximum(bt1_, cmax)
                nb2 = jnp.maximum(jnp.minimum(bt1_, cmax),
                                  jnp.maximum(bt2_, cmax2))
                nbd = jnp.where(cmax > bt1_, dsel, bd_)
                return nb1, nb2, nbd

            def p1_body(i, carry):
                out = []
                for q in range(4):
                    t1_, t2_, d_ = carry[3 * q:3 * q + 3]
                    out.extend(chunk_upd((4 * i + q) * C1, t1_, t2_, d_))
                return tuple(out)

            neg = jnp.full((rows, 1), _NEG_INF, jnp.float32)
            zero = jnp.zeros((rows, 1), jnp.int32)
            acc = jax.lax.fori_loop(
                0, N1P, p1_body, (neg, neg, zero) * 4)

            def merge(a1, a2, ad, b1, b2, bd_):
                m1 = jnp.maximum(a1, b1)
                m2 = jnp.maximum(jnp.minimum(a1, b1), jnp.maximum(a2, b2))
                md = jnp.where(a1 >= b1, ad, bd_)
                return m1, m2, md

            mA = merge(*acc[0:6])
            mB = merge(*acc[6:12])
            return merge(*mA, *mB)

        if s == 0:
            cn2 = jnp.concatenate([cn[0:1], cn[7:8]], axis=0)
            ct2 = jnp.concatenate([ct[0:1], ct[7:8]], axis=0)
            b1_2, b2_2, bd_2 = scan_top2(cn2, ct2)
            sel = (wk // 10) == ((8 * g) // 10)
            tmax_raw = jnp.where(sel, b1_2[0:1], b1_2[1:2])
            t2_raw = jnp.where(sel, b2_2[0:1], b2_2[1:2])
            bd1 = jnp.where(sel, bd_2[0:1], bd_2[1:2])
        else:
            tmax_raw, t2_raw, bd1 = scan_top2(cn, ct)
        has_valid = tmax_raw > _NEG_INF
        tmax = jnp.where(has_valid, tmax_raw, jnp.float32(0.0))
        multi = jnp.any((t2_raw >= tmax_raw - jnp.float32(3.0))
                        & (t2_raw > _NEG_INF))

        def p2_body(i, carry):
            bv, bd, bt = carry
            src = src_ref[:, pl.ds(i * C2, C2)]
            dst = dst_ref[:, pl.ds(i * C2, C2)]
            tt = t_ref[:, pl.ds(i * C2, C2)]
            valid = (src == cn) & (tt < ct)
            j = i * C2 + jax.lax.broadcasted_iota(jnp.int32, (1, C2), 1)
            idxs = (wk * E_ALL + j).astype(jnp.uint32)
            gmb = _gumbel_from_bits(_tf_bits(idxs, k0, k1))
            logits = jnp.where(valid, (tt - tmax) / jnp.float32(TEMPERATURE),
                               _NEG_INF)
            score = logits + gmb
            cmax = jnp.max(score, axis=1, keepdims=True)
            jbig = jnp.where(score == cmax, j, jnp.int32(2**31 - 1))
            jsel = jnp.min(jbig, axis=1, keepdims=True)
            onehot = j == jsel
            dsel = jnp.sum(jnp.where(onehot, dst, 0), axis=1, keepdims=True)
            tsel = jnp.sum(jnp.where(onehot, tt, jnp.float32(0.0)),
                           axis=1, keepdims=True)
            upd = cmax > bv
            return (jnp.where(upd, cmax, bv),
                    jnp.where(upd, dsel, bd),
                    jnp.where(upd, tsel, bt))

        def full_path():
            _, bd2, bt2 = jax.lax.fori_loop(
                0, N2, p2_body,
                (jnp.full((8, 1), _NEG_INF, jnp.float32),
                 jnp.zeros((8, 1), jnp.int32),
                 jnp.zeros((8, 1), jnp.float32)))
            return bd2, bt2

        def fast_path():
            return bd1, tmax_raw

        bd, bt = jax.lax.cond(multi, full_path, fast_path)

        alive = alive & has_valid
        cn = jnp.where(alive, bd, cn)
        ct = jnp.where(alive, bt, ct)
        on_ref[0, :, s + 1:s + 2] = jnp.where(alive, bd, 0)
        ot_ref[0, :, s + 1:s + 2] = jnp.where(alive, bt, jnp.float32(0.0))
        om_ref[0, :, s + 1:s + 2] = alive.astype(jnp.float32)


def _restart_kernel(sn_ref, ct_ref, mem_ref, w_ref, b_ref, tw_ref, tb_ref,
                    out_ref, mrows_ref):
    def gather_body(i, _):
        idx = sn_ref[i]
        mrows_ref[pl.ds(i, 1), :] = mem_ref[pl.ds(idx, 1), :]
        return 0

    jax.lax.fori_loop(0, BATCH, gather_body, 0)
    mem = mrows_ref[...]
    te = jnp.cos(ct_ref[...] * tw_ref[...] + tb_ref[...])
    wm = w_ref[:, :MEMORY_DIM]
    wt = w_ref[:, MEMORY_DIM:]
    r = (jnp.sum(mem * wm, axis=1, keepdims=True)
         + jnp.sum(te * wt, axis=1, keepdims=True) + b_ref[...])
    out_ref[...] = jax.nn.sigmoid(r)


def kernel(source_nodes, current_times, edge_index, edge_time, memory_states,
           W_restart, b_restart, time_w, time_b):
    src_all = jnp.concatenate([edge_index[0], edge_index[1]]).astype(jnp.int32)
    dst_all = jnp.concatenate([edge_index[1], edge_index[0]]).astype(jnp.int32)
    t_all = jnp.concatenate([edge_time, edge_time]).astype(jnp.float32)
    pad = E_PAD - E_ALL
    src_p = jnp.pad(src_all, (0, pad), constant_values=-1)[None, :]
    dst_p = jnp.pad(dst_all, (0, pad), constant_values=0)[None, :]
    t_p = jnp.pad(t_all, (0, pad), constant_values=0.0)[None, :]

    n0 = jnp.broadcast_to(source_nodes.astype(jnp.int32)[:, None],
                          (BATCH, NUM_WALKS)).reshape(GROUPS, 8, 1)
    t0 = jnp.broadcast_to(current_times.astype(jnp.float32)[:, None],
                          (BATCH, NUM_WALKS)).reshape(GROUPS, 8, 1)

    full = pl.BlockSpec((1, E_PAD), lambda g: (0, 0))
    state = pl.BlockSpec((1, 8, 1), lambda g: (g, 0, 0))
    out3 = pl.BlockSpec((1, 8, WALK_LEN), lambda g: (g, 0, 0))

    on, ot, om = pl.pallas_call(
        _walks_kernel,
        grid=(GROUPS,),
        in_specs=[full, full, full, state, state],
        out_specs=[out3, out3, out3],
        out_shape=[
            jax.ShapeDtypeStruct((GROUPS, 8, WALK_LEN), jnp.int32),
            jax.ShapeDtypeStruct((GROUPS, 8, WALK_LEN), jnp.float32),
            jax.ShapeDtypeStruct((GROUPS, 8, WALK_LEN), jnp.float32),
        ],
    )(src_p, dst_p, t_p, n0, t0)

    walk_nodes = on.reshape(BATCH, NUM_WALKS, WALK_LEN)
    walk_times = ot.reshape(BATCH, NUM_WALKS, WALK_LEN)
    walk_masks = om.reshape(BATCH, NUM_WALKS, WALK_LEN)

    restart_probs = pl.pallas_call(
        _restart_kernel,
        in_specs=[
            pl.BlockSpec(memory_space=pltpu.SMEM),
            pl.BlockSpec((BATCH, 1), lambda: (0, 0)),
            pl.BlockSpec((NUM_NODES, MEMORY_DIM), lambda: (0, 0)),
            pl.BlockSpec((1, MEMORY_DIM + TIME_DIM), lambda: (0, 0)),
            pl.BlockSpec((1, 1), lambda: (0, 0)),
            pl.BlockSpec((1, TIME_DIM), lambda: (0, 0)),
            pl.BlockSpec((1, TIME_DIM), lambda: (0, 0)),
        ],
        out_specs=pl.BlockSpec((BATCH, 1), lambda: (0, 0)),
        out_shape=jax.ShapeDtypeStruct((BATCH, 1), jnp.float32),
        scratch_shapes=[pltpu.VMEM((BATCH, MEMORY_DIM), jnp.float32)],
    )(source_nodes.astype(jnp.int32),
      current_times.astype(jnp.float32)[:, None],
      memory_states.astype(jnp.float32),
      W_restart.astype(jnp.float32).reshape(1, -1),
      b_restart.astype(jnp.float32).reshape(1, 1),
      time_w.astype(jnp.float32)[None, :],
      time_b.astype(jnp.float32)[None, :])

    return walk_nodes, walk_times, walk_masks, restart_probs


# speedup vs baseline: 28.3061x; 1.1302x over previous
import numpy as np
import jax
import jax.numpy as jnp
from jax.experimental import pallas as pl
from jax.experimental.pallas import tpu as pltpu

NUM_NODES = 10000
NUM_EDGES = 50000
BATCH = 32
MEMORY_DIM = 128
TIME_DIM = 64
NUM_WALKS = 10
WALK_LEN = 3
TEMPERATURE = 0.1

E_ALL = 2 * NUM_EDGES
W_TOT = BATCH * NUM_WALKS
GROUPS = W_TOT // 8
C1 = 6272
C2 = 1024
N1P = 2
N2 = 98
E_PAD = 8 * N1P * C1

_TINY = np.float32(np.finfo(np.float32).tiny)
_NEG_INF = np.float32(-np.inf)


def _np_threefry_pair(k0, k1, x0, x1):
    x0 = np.asarray(x0, np.uint32).copy()
    x1 = np.asarray(x1, np.uint32).copy()
    ks0 = np.uint32(k0)
    ks1 = np.uint32(k1)
    ks2 = np.uint32(ks0 ^ ks1 ^ np.uint32(0x1BD11BDA))
    rots = ((13, 15, 26, 6), (17, 29, 16, 24))
    sched = ((ks1, ks2), (ks2, ks0), (ks0, ks1), (ks1, ks2), (ks2, ks0))
    x0 = (x0 + ks0).astype(np.uint32)
    x1 = (x1 + ks1).astype(np.uint32)
    for i in range(5):
        for r in rots[i % 2]:
            x0 = (x0 + x1).astype(np.uint32)
            x1 = ((x1 << np.uint32(r)) | (x1 >> np.uint32(32 - r))).astype(np.uint32)
            x1 = (x1 ^ x0).astype(np.uint32)
        a, b = sched[i]
        x0 = (x0 + a).astype(np.uint32)
        x1 = (x1 + b + np.uint32(i + 1)).astype(np.uint32)
    return x0, x1


def _derive_step_keys():
    keys = []
    k = (np.uint32(0), np.uint32(1234))
    for _ in range(WALK_LEN - 1):
        h0, h1 = _np_threefry_pair(k[0], k[1], np.zeros(2, np.uint32),
                                   np.arange(2, dtype=np.uint32))
        k = (h0[0], h1[0])
        keys.append((int(h0[1]), int(h1[1])))
    return keys


_STEP_KEYS = _derive_step_keys()


def _tf_bits(x1, k0, k1):
    ks0 = jnp.uint32(k0)
    ks1 = jnp.uint32(k1)
    ks2 = jnp.uint32(k0 ^ k1 ^ 0x1BD11BDA)
    rots = ((13, 15, 26, 6), (17, 29, 16, 24))
    sched = ((ks1, ks2), (ks2, ks0), (ks0, ks1), (ks1, ks2), (ks2, ks0))
    x0 = jnp.full_like(x1, ks0)
    x1 = x1 + ks1
    for i in range(5):
        for r in rots[i % 2]:
            x0 = x0 + x1
            x1 = (x1 << jnp.uint32(r)) | (x1 >> jnp.uint32(32 - r))
            x1 = x1 ^ x0
        a, b = sched[i]
        x0 = x0 + a
        x1 = x1 + b + jnp.uint32(i + 1)
    return x0 ^ x1


def _gumbel_from_bits(bits):
    fb = (bits >> jnp.uint32(9)) | jnp.uint32(0x3F800000)
    f = jax.lax.bitcast_convert_type(fb, jnp.float32) - jnp.float32(1.0)
    u = jnp.maximum(_TINY, f * (jnp.float32(1.0) - _TINY) + _TINY)
    return -jnp.log(-jnp.log(u))


def _walks_kernel(src_ref, dst_ref, t_ref, n0_ref, t0_ref,
                  on_ref, ot_ref, om_ref):
    g = pl.program_id(0)
    cn = n0_ref[0]
    ct = t0_ref[0]
    alive = jnp.ones((8, 1), dtype=jnp.bool_)
    wk = jax.lax.broadcasted_iota(jnp.int32, (8, 1), 0) + 8 * g

    on_ref[0, :, 0:1] = cn
    ot_ref[0, :, 0:1] = ct
    om_ref[0, :, 0:1] = jnp.ones((8, 1), jnp.float32)

    for s in range(WALK_LEN - 1):
        k0, k1 = _STEP_KEYS[s]

        def scan_top2(cnx, ctx):
            rows = cnx.shape[0]

            def chunk_upd(off, bt1_, bt2_, bd_):
                src = src_ref[:, pl.ds(off, C1)]
                dst = dst_ref[:, pl.ds(off, C1)]
                tt = t_ref[:, pl.ds(off, C1)]
                valid = (src == cnx) & (tt < ctx)
                tc = jnp.where(valid, tt, _NEG_INF)
                cmax = jnp.max(tc, axis=1, keepdims=True)
                j = jax.lax.broadcasted_iota(jnp.int32, (1, C1), 1)
                jbig = jnp.where(tc == cmax, j, jnp.int32(2**31 - 1))
                jsel = jnp.min(jbig, axis=1, keepdims=True)
                onehot = j == jsel
                dsel = jnp.sum(jnp.where(onehot, dst, 0),
                               axis=1, keepdims=True)
                cmax2 = jnp.max(jnp.where(onehot, _NEG_INF, tc),
                                axis=1, keepdims=True)
                nb1 = jnp.maximum(bt1_, cmax)
                nb2 = jnp.maximum(jnp.minimum(bt1_, cmax),
                                  jnp.maximum(bt2_, cmax2))
                nbd = jnp.where(cmax > bt1_, dsel, bd_)
                return nb1, nb2, nbd

            def p1_body(i, carry):
                out = []
                for q in range(8):
                    t1_, t2_, d_ = carry[3 * q:3 * q + 3]
                    out.extend(chunk_upd((8 * i + q) * C1, t1_, t2_, d_))
                return tuple(out)

            neg = jnp.full((rows, 1), _NEG_INF, jnp.float32)
            zero = jnp.zeros((rows, 1), jnp.int32)
            acc = jax.lax.fori_loop(
                0, N1P, p1_body, (neg, neg, zero) * 8)

            def merge(a1, a2, ad, b1, b2, bd_):
                m1 = jnp.maximum(a1, b1)
                m2 = jnp.maximum(jnp.minimum(a1, b1), jnp.maximum(a2, b2))
                md = jnp.where(a1 >= b1, ad, bd_)
                return m1, m2, md

            m = [merge(*acc[6 * q:6 * q + 6]) for q in range(4)]
            mA = merge(*m[0], *m[1])
            mB = merge(*m[2], *m[3])
            return merge(*mA, *mB)

        if s == 0:
            cn2 = jnp.concatenate([cn[0:1], cn[7:8]], axis=0)
            ct2 = jnp.concatenate([ct[0:1], ct[7:8]], axis=0)
            b1_2, b2_2, bd_2 = scan_top2(cn2, ct2)
            sel = (wk // 10) == ((8 * g) // 10)
            tmax_raw = jnp.where(sel, b1_2[0:1], b1_2[1:2])
            t2_raw = jnp.where(sel, b2_2[0:1], b2_2[1:2])
            bd1 = jnp.where(sel, bd_2[0:1], bd_2[1:2])
        else:
            tmax_raw, t2_raw, bd1 = scan_top2(cn, ct)
        has_valid = tmax_raw > _NEG_INF
        tmax = jnp.where(has_valid, tmax_raw, jnp.float32(0.0))
        multi = jnp.any((t2_raw >= tmax_raw - jnp.float32(3.0))
                        & (t2_raw > _NEG_INF))

        def p2_body(i, carry):
            bv, bd, bt = carry
            src = src_ref[:, pl.ds(i * C2, C2)]
            dst = dst_ref[:, pl.ds(i * C2, C2)]
            tt = t_ref[:, pl.ds(i * C2, C2)]
            valid = (src == cn) & (tt < ct)
            j = i * C2 + jax.lax.broadcasted_iota(jnp.int32, (1, C2), 1)
            idxs = (wk * E_ALL + j).astype(jnp.uint32)
            gmb = _gumbel_from_bits(_tf_bits(idxs, k0, k1))
            logits = jnp.where(valid, (tt - tmax) / jnp.float32(TEMPERATURE),
                               _NEG_INF)
            score = logits + gmb
            cmax = jnp.max(score, axis=1, keepdims=True)
            jbig = jnp.where(score == cmax, j, jnp.int32(2**31 - 1))
            jsel = jnp.min(jbig, axis=1, keepdims=True)
            onehot = j == jsel
            dsel = jnp.sum(jnp.where(onehot, dst, 0), axis=1, keepdims=True)
            tsel = jnp.sum(jnp.where(onehot, tt, jnp.float32(0.0)),
                           axis=1, keepdims=True)
            upd = cmax > bv
            return (jnp.where(upd, cmax, bv),
                    jnp.where(upd, dsel, bd),
                    jnp.where(upd, tsel, bt))

        def full_path():
            _, bd2, bt2 = jax.lax.fori_loop(
                0, N2, p2_body,
                (jnp.full((8, 1), _NEG_INF, jnp.float32),
                 jnp.zeros((8, 1), jnp.int32),
                 jnp.zeros((8, 1), jnp.float32)))
            return bd2, bt2

        def fast_path():
            return bd1, tmax_raw

        bd, bt = jax.lax.cond(multi, full_path, fast_path)

        alive = alive & has_valid
        cn = jnp.where(alive, bd, cn)
        ct = jnp.where(alive, bt, ct)
        on_ref[0, :, s + 1:s + 2] = jnp.where(alive, bd, 0)
        ot_ref[0, :, s + 1:s + 2] = jnp.where(alive, bt, jnp.float32(0.0))
        om_ref[0, :, s + 1:s + 2] = alive.astype(jnp.float32)


def _restart_kernel(sn_ref, ct_ref, mem_ref, w_ref, b_ref, tw_ref, tb_ref,
                    out_ref, mrows_ref):
    def gather_body(i, _):
        idx = sn_ref[i]
        mrows_ref[pl.ds(i, 1), :] = mem_ref[pl.ds(idx, 1), :]
        return 0

    jax.lax.fori_loop(0, BATCH, gather_body, 0)
    mem = mrows_ref[...]
    te = jnp.cos(ct_ref[...] * tw_ref[...] + tb_ref[...])
    wm = w_ref[:, :MEMORY_DIM]
    wt = w_ref[:, MEMORY_DIM:]
    r = (jnp.sum(mem * wm, axis=1, keepdims=True)
         + jnp.sum(te * wt, axis=1, keepdims=True) + b_ref[...])
    out_ref[...] = jax.nn.sigmoid(r)


def kernel(source_nodes, current_times, edge_index, edge_time, memory_states,
           W_restart, b_restart, time_w, time_b):
    src_all = jnp.concatenate([edge_index[0], edge_index[1]]).astype(jnp.int32)
    dst_all = jnp.concatenate([edge_index[1], edge_index[0]]).astype(jnp.int32)
    t_all = jnp.concatenate([edge_time, edge_time]).astype(jnp.float32)
    pad = E_PAD - E_ALL
    src_p = jnp.pad(src_all, (0, pad), constant_values=-1)[None, :]
    dst_p = jnp.pad(dst_all, (0, pad), constant_values=0)[None, :]
    t_p = jnp.pad(t_all, (0, pad), constant_values=0.0)[None, :]

    n0 = jnp.broadcast_to(source_nodes.astype(jnp.int32)[:, None],
                          (BATCH, NUM_WALKS)).reshape(GROUPS, 8, 1)
    t0 = jnp.broadcast_to(current_times.astype(jnp.float32)[:, None],
                          (BATCH, NUM_WALKS)).reshape(GROUPS, 8, 1)

    full = pl.BlockSpec((1, E_PAD), lambda g: (0, 0))
    state = pl.BlockSpec((1, 8, 1), lambda g: (g, 0, 0))
    out3 = pl.BlockSpec((1, 8, WALK_LEN), lambda g: (g, 0, 0))

    on, ot, om = pl.pallas_call(
        _walks_kernel,
        grid=(GROUPS,),
        in_specs=[full, full, full, state, state],
        out_specs=[out3, out3, out3],
        out_shape=[
            jax.ShapeDtypeStruct((GROUPS, 8, WALK_LEN), jnp.int32),
            jax.ShapeDtypeStruct((GROUPS, 8, WALK_LEN), jnp.float32),
            jax.ShapeDtypeStruct((GROUPS, 8, WALK_LEN), jnp.float32),
        ],
    )(src_p, dst_p, t_p, n0, t0)

    walk_nodes = on.reshape(BATCH, NUM_WALKS, WALK_LEN)
    walk_times = ot.reshape(BATCH, NUM_WALKS, WALK_LEN)
    walk_masks = om.reshape(BATCH, NUM_WALKS, WALK_LEN)

    restart_probs = pl.pallas_call(
        _restart_kernel,
        in_specs=[
            pl.BlockSpec(memory_space=pltpu.SMEM),
            pl.BlockSpec((BATCH, 1), lambda: (0, 0)),
            pl.BlockSpec((NUM_NODES, MEMORY_DIM), lambda: (0, 0)),
            pl.BlockSpec((1, MEMORY_DIM + TIME_DIM), lambda: (0, 0)),
            pl.BlockSpec((1, 1), lambda: (0, 0)),
            pl.BlockSpec((1, TIME_DIM), lambda: (0, 0)),
            pl.BlockSpec((1, TIME_DIM), lambda: (0, 0)),
        ],
        out_specs=pl.BlockSpec((BATCH, 1), lambda: (0, 0)),
        out_shape=jax.ShapeDtypeStruct((BATCH, 1), jnp.float32),
        scratch_shapes=[pltpu.VMEM((BATCH, MEMORY_DIM), jnp.float32)],
    )(source_nodes.astype(jnp.int32),
      current_times.astype(jnp.float32)[:, None],
      memory_states.astype(jnp.float32),
      W_restart.astype(jnp.float32).reshape(1, -1),
      b_restart.astype(jnp.float32).reshape(1, 1),
      time_w.astype(jnp.float32)[None, :],
      time_b.astype(jnp.float32)[None, :])

    return walk_nodes, walk_times, walk_masks, restart_probs


# speedup vs baseline: 30.4981x; 1.0774x over previous
import numpy as np
import jax
import jax.numpy as jnp
from jax.experimental import pallas as pl
from jax.experimental.pallas import tpu as pltpu

NUM_NODES = 10000
NUM_EDGES = 50000
BATCH = 32
MEMORY_DIM = 128
TIME_DIM = 64
NUM_WALKS = 10
WALK_LEN = 3
TEMPERATURE = 0.1

E_ALL = 2 * NUM_EDGES
W_TOT = BATCH * NUM_WALKS
GROUPS = W_TOT // 8
C1 = 6272
C2 = 1024
N1 = 16
N2 = 98
E_PAD = N1 * C1

_TINY = np.float32(np.finfo(np.float32).tiny)
_NEG_INF = np.float32(-np.inf)


def _np_threefry_pair(k0, k1, x0, x1):
    x0 = np.asarray(x0, np.uint32).copy()
    x1 = np.asarray(x1, np.uint32).copy()
    ks0 = np.uint32(k0)
    ks1 = np.uint32(k1)
    ks2 = np.uint32(ks0 ^ ks1 ^ np.uint32(0x1BD11BDA))
    rots = ((13, 15, 26, 6), (17, 29, 16, 24))
    sched = ((ks1, ks2), (ks2, ks0), (ks0, ks1), (ks1, ks2), (ks2, ks0))
    x0 = (x0 + ks0).astype(np.uint32)
    x1 = (x1 + ks1).astype(np.uint32)
    for i in range(5):
        for r in rots[i % 2]:
            x0 = (x0 + x1).astype(np.uint32)
            x1 = ((x1 << np.uint32(r)) | (x1 >> np.uint32(32 - r))).astype(np.uint32)
            x1 = (x1 ^ x0).astype(np.uint32)
        a, b = sched[i]
        x0 = (x0 + a).astype(np.uint32)
        x1 = (x1 + b + np.uint32(i + 1)).astype(np.uint32)
    return x0, x1


def _derive_step_keys():
    keys = []
    k = (np.uint32(0), np.uint32(1234))
    for _ in range(WALK_LEN - 1):
        h0, h1 = _np_threefry_pair(k[0], k[1], np.zeros(2, np.uint32),
                                   np.arange(2, dtype=np.uint32))
        k = (h0[0], h1[0])
        keys.append((int(h0[1]), int(h1[1])))
    return keys


_STEP_KEYS = _derive_step_keys()


def _tf_bits(x1, k0, k1):
    ks0 = jnp.uint32(k0)
    ks1 = jnp.uint32(k1)
    ks2 = jnp.uint32(k0 ^ k1 ^ 0x1BD11BDA)
    rots = ((13, 15, 26, 6), (17, 29, 16, 24))
    sched = ((ks1, ks2), (ks2, ks0), (ks0, ks1), (ks1, ks2), (ks2, ks0))
    x0 = jnp.full_like(x1, ks0)
    x1 = x1 + ks1
    for i in range(5):
        for r in rots[i % 2]:
            x0 = x0 + x1
            x1 = (x1 << jnp.uint32(r)) | (x1 >> jnp.uint32(32 - r))
            x1 = x1 ^ x0
        a, b = sched[i]
        x0 = x0 + a
        x1 = x1 + b + jnp.uint32(i + 1)
    return x0 ^ x1


def _gumbel_from_bits(bits):
    fb = (bits >> jnp.uint32(9)) | jnp.uint32(0x3F800000)
    f = jax.lax.bitcast_convert_type(fb, jnp.float32) - jnp.float32(1.0)
    u = jnp.maximum(_TINY, f * (jnp.float32(1.0) - _TINY) + _TINY)
    return -jnp.log(-jnp.log(u))


def _walks_kernel(src_ref, dst_ref, t_ref, n0_ref, t0_ref,
                  on_ref, ot_ref, om_ref):
    g = pl.program_id(0)
    cn = n0_ref[0]
    ct = t0_ref[0]
    alive = jnp.ones((8, 1), dtype=jnp.bool_)
    wk = jax.lax.broadcasted_iota(jnp.int32, (8, 1), 0) + 8 * g

    on_ref[0, :, 0:1] = cn
    ot_ref[0, :, 0:1] = ct
    om_ref[0, :, 0:1] = jnp.ones((8, 1), jnp.float32)

    for s in range(WALK_LEN - 1):
        k0, k1 = _STEP_KEYS[s]

        def scan_top2(cnx, ctx):
            rows = cnx.shape[0]

            def chunk_upd(off, bt1_, bt2_, bd_):
                src = src_ref[:, pl.ds(off, C1)]
                dst = dst_ref[:, pl.ds(off, C1)]
                tt = t_ref[:, pl.ds(off, C1)]
                valid = (src == cnx) & (tt < ctx)
                tc = jnp.where(valid, tt, _NEG_INF)
                cmax = jnp.max(tc, axis=1, keepdims=True)
                j = jax.lax.broadcasted_iota(jnp.int32, (1, C1), 1)
                jbig = jnp.where(tc == cmax, j, jnp.int32(2**31 - 1))
                jsel = jnp.min(jbig, axis=1, keepdims=True)
                onehot = j == jsel
                dsel = jnp.sum(jnp.where(onehot, dst, 0),
                               axis=1, keepdims=True)
                cmax2 = jnp.max(jnp.where(onehot, _NEG_INF, tc),
                                axis=1, keepdims=True)
                nb1 = jnp.maximum(bt1_, cmax)
                nb2 = jnp.maximum(jnp.minimum(bt1_, cmax),
                                  jnp.maximum(bt2_, cmax2))
                nbd = jnp.where(cmax > bt1_, dsel, bd_)
                return nb1, nb2, nbd

            neg = jnp.full((rows, 1), _NEG_INF, jnp.float32)
            zero = jnp.zeros((rows, 1), jnp.int32)
            acc = []
            for q in range(N1):
                acc.extend(chunk_upd(q * C1, neg, neg, zero))
            acc = tuple(acc)

            def merge(a1, a2, ad, b1, b2, bd_):
                m1 = jnp.maximum(a1, b1)
                m2 = jnp.maximum(jnp.minimum(a1, b1), jnp.maximum(a2, b2))
                md = jnp.where(a1 >= b1, ad, bd_)
                return m1, m2, md

            m = [merge(*acc[6 * q:6 * q + 6]) for q in range(N1 // 2)]
            while len(m) > 1:
                m = [merge(*m[2 * q], *m[2 * q + 1])
                     for q in range(len(m) // 2)]
            return m[0]

        if s == 0:
            cn2 = jnp.concatenate([cn[0:1], cn[7:8]], axis=0)
            ct2 = jnp.concatenate([ct[0:1], ct[7:8]], axis=0)
            b1_2, b2_2, bd_2 = scan_top2(cn2, ct2)
            sel = (wk // 10) == ((8 * g) // 10)
            tmax_raw = jnp.where(sel, b1_2[0:1], b1_2[1:2])
            t2_raw = jnp.where(sel, b2_2[0:1], b2_2[1:2])
            bd1 = jnp.where(sel, bd_2[0:1], bd_2[1:2])
        else:
            tmax_raw, t2_raw, bd1 = scan_top2(cn, ct)
        has_valid = tmax_raw > _NEG_INF
        tmax = jnp.where(has_valid, tmax_raw, jnp.float32(0.0))
        multi = jnp.any((t2_raw >= tmax_raw - jnp.float32(3.0))
                        & (t2_raw > _NEG_INF))

        def p2_body(i, carry):
            bv, bd, bt = carry
            src = src_ref[:, pl.ds(i * C2, C2)]
            dst = dst_ref[:, pl.ds(i * C2, C2)]
            tt = t_ref[:, pl.ds(i * C2, C2)]
            valid = (src == cn) & (tt < ct)
            j = i * C2 + jax.lax.broadcasted_iota(jnp.int32, (1, C2), 1)
            idxs = (wk * E_ALL + j).astype(jnp.uint32)
            gmb = _gumbel_from_bits(_tf_bits(idxs, k0, k1))
            logits = jnp.where(valid, (tt - tmax) / jnp.float32(TEMPERATURE),
                               _NEG_INF)
            score = logits + gmb
            cmax = jnp.max(score, axis=1, keepdims=True)
            jbig = jnp.where(score == cmax, j, jnp.int32(2**31 - 1))
            jsel = jnp.min(jbig, axis=1, keepdims=True)
            onehot = j == jsel
            dsel = jnp.sum(jnp.where(onehot, dst, 0), axis=1, keepdims=True)
            tsel = jnp.sum(jnp.where(onehot, tt, jnp.float32(0.0)),
                           axis=1, keepdims=True)
            upd = cmax > bv
            return (jnp.where(upd, cmax, bv),
                    jnp.where(upd, dsel, bd),
                    jnp.where(upd, tsel, bt))

        def full_path():
            _, bd2, bt2 = jax.lax.fori_loop(
                0, N2, p2_body,
                (jnp.full((8, 1), _NEG_INF, jnp.float32),
                 jnp.zeros((8, 1), jnp.int32),
                 jnp.zeros((8, 1), jnp.float32)))
            return bd2, bt2

        def fast_path():
            return bd1, tmax_raw

        bd, bt = jax.lax.cond(multi, full_path, fast_path)

        alive = alive & has_valid
        cn = jnp.where(alive, bd, cn)
        ct = jnp.where(alive, bt, ct)
        on_ref[0, :, s + 1:s + 2] = jnp.where(alive, bd, 0)
        ot_ref[0, :, s + 1:s + 2] = jnp.where(alive, bt, jnp.float32(0.0))
        om_ref[0, :, s + 1:s + 2] = alive.astype(jnp.float32)


def _restart_kernel(sn_ref, ct_ref, mem_ref, w_ref, b_ref, tw_ref, tb_ref,
                    out_ref, mrows_ref):
    def gather_body(i, _):
        idx = sn_ref[i]
        mrows_ref[pl.ds(i, 1), :] = mem_ref[pl.ds(idx, 1), :]
        return 0

    jax.lax.fori_loop(0, BATCH, gather_body, 0)
    mem = mrows_ref[...]
    te = jnp.cos(ct_ref[...] * tw_ref[...] + tb_ref[...])
    wm = w_ref[:, :MEMORY_DIM]
    wt = w_ref[:, MEMORY_DIM:]
    r = (jnp.sum(mem * wm, axis=1, keepdims=True)
         + jnp.sum(te * wt, axis=1, keepdims=True) + b_ref[...])
    out_ref[...] = jax.nn.sigmoid(r)


def kernel(source_nodes, current_times, edge_index, edge_time, memory_states,
           W_restart, b_restart, time_w, time_b):
    src_all = jnp.concatenate([edge_index[0], edge_index[1]]).astype(jnp.int32)
    dst_all = jnp.concatenate([edge_index[1], edge_index[0]]).astype(jnp.int32)
    t_all = jnp.concatenate([edge_time, edge_time]).astype(jnp.float32)
    pad = E_PAD - E_ALL
    src_p = jnp.pad(src_all, (0, pad), constant_values=-1)[None, :]
    dst_p = jnp.pad(dst_all, (0, pad), constant_values=0)[None, :]
    t_p = jnp.pad(t_all, (0, pad), constant_values=0.0)[None, :]

    n0 = jnp.broadcast_to(source_nodes.astype(jnp.int32)[:, None],
                          (BATCH, NUM_WALKS)).reshape(GROUPS, 8, 1)
    t0 = jnp.broadcast_to(current_times.astype(jnp.float32)[:, None],
                          (BATCH, NUM_WALKS)).reshape(GROUPS, 8, 1)

    full = pl.BlockSpec((1, E_PAD), lambda g: (0, 0))
    state = pl.BlockSpec((1, 8, 1), lambda g: (g, 0, 0))
    out3 = pl.BlockSpec((1, 8, WALK_LEN), lambda g: (g, 0, 0))

    on, ot, om = pl.pallas_call(
        _walks_kernel,
        grid=(GROUPS,),
        in_specs=[full, full, full, state, state],
        out_specs=[out3, out3, out3],
        out_shape=[
            jax.ShapeDtypeStruct((GROUPS, 8, WALK_LEN), jnp.int32),
            jax.ShapeDtypeStruct((GROUPS, 8, WALK_LEN), jnp.float32),
            jax.ShapeDtypeStruct((GROUPS, 8, WALK_LEN), jnp.float32),
        ],
    )(src_p, dst_p, t_p, n0, t0)

    walk_nodes = on.reshape(BATCH, NUM_WALKS, WALK_LEN)
    walk_times = ot.reshape(BATCH, NUM_WALKS, WALK_LEN)
    walk_masks = om.reshape(BATCH, NUM_WALKS, WALK_LEN)

    restart_probs = pl.pallas_call(
        _restart_kernel,
        in_specs=[
            pl.BlockSpec(memory_space=pltpu.SMEM),
            pl.BlockSpec((BATCH, 1), lambda: (0, 0)),
            pl.BlockSpec((NUM_NODES, MEMORY_DIM), lambda: (0, 0)),
            pl.BlockSpec((1, MEMORY_DIM + TIME_DIM), lambda: (0, 0)),
            pl.BlockSpec((1, 1), lambda: (0, 0)),
            pl.BlockSpec((1, TIME_DIM), lambda: (0, 0)),
            pl.BlockSpec((1, TIME_DIM), lambda: (0, 0)),
        ],
        out_specs=pl.BlockSpec((BATCH, 1), lambda: (0, 0)),
        out_shape=jax.ShapeDtypeStruct((BATCH, 1), jnp.float32),
        scratch_shapes=[pltpu.VMEM((BATCH, MEMORY_DIM), jnp.float32)],
    )(source_nodes.astype(jnp.int32),
      current_times.astype(jnp.float32)[:, None],
      memory_states.astype(jnp.float32),
      W_restart.astype(jnp.float32).reshape(1, -1),
      b_restart.astype(jnp.float32).reshape(1, 1),
      time_w.astype(jnp.float32)[None, :],
      time_b.astype(jnp.float32)[None, :])

    return walk_nodes, walk_times, walk_masks, restart_probs


# speedup vs baseline: 38.4658x; 1.2613x over previous
import numpy as np
import jax
import jax.numpy as jnp
from jax.experimental import pallas as pl
from jax.experimental.pallas import tpu as pltpu

NUM_NODES = 10000
NUM_EDGES = 50000
BATCH = 32
MEMORY_DIM = 128
TIME_DIM = 64
NUM_WALKS = 10
WALK_LEN = 3
TEMPERATURE = 0.1

E_ALL = 2 * NUM_EDGES
W_TOT = BATCH * NUM_WALKS
GROUPS = W_TOT // 8
C1 = 6272
C2 = 1024
N1 = 16
N2 = 98
E_PAD = N1 * C1

_TINY = np.float32(np.finfo(np.float32).tiny)
_NEG_INF = np.float32(-np.inf)


def _np_threefry_pair(k0, k1, x0, x1):
    x0 = np.asarray(x0, np.uint32).copy()
    x1 = np.asarray(x1, np.uint32).copy()
    ks0 = np.uint32(k0)
    ks1 = np.uint32(k1)
    ks2 = np.uint32(ks0 ^ ks1 ^ np.uint32(0x1BD11BDA))
    rots = ((13, 15, 26, 6), (17, 29, 16, 24))
    sched = ((ks1, ks2), (ks2, ks0), (ks0, ks1), (ks1, ks2), (ks2, ks0))
    x0 = (x0 + ks0).astype(np.uint32)
    x1 = (x1 + ks1).astype(np.uint32)
    for i in range(5):
        for r in rots[i % 2]:
            x0 = (x0 + x1).astype(np.uint32)
            x1 = ((x1 << np.uint32(r)) | (x1 >> np.uint32(32 - r))).astype(np.uint32)
            x1 = (x1 ^ x0).astype(np.uint32)
        a, b = sched[i]
        x0 = (x0 + a).astype(np.uint32)
        x1 = (x1 + b + np.uint32(i + 1)).astype(np.uint32)
    return x0, x1


def _derive_step_keys():
    keys = []
    k = (np.uint32(0), np.uint32(1234))
    for _ in range(WALK_LEN - 1):
        h0, h1 = _np_threefry_pair(k[0], k[1], np.zeros(2, np.uint32),
                                   np.arange(2, dtype=np.uint32))
        k = (h0[0], h1[0])
        keys.append((int(h0[1]), int(h1[1])))
    return keys


_STEP_KEYS = _derive_step_keys()


def _tf_bits(x1, k0, k1):
    ks0 = jnp.uint32(k0)
    ks1 = jnp.uint32(k1)
    ks2 = jnp.uint32(k0 ^ k1 ^ 0x1BD11BDA)
    rots = ((13, 15, 26, 6), (17, 29, 16, 24))
    sched = ((ks1, ks2), (ks2, ks0), (ks0, ks1), (ks1, ks2), (ks2, ks0))
    x0 = jnp.full_like(x1, ks0)
    x1 = x1 + ks1
    for i in range(5):
        for r in rots[i % 2]:
            x0 = x0 + x1
            x1 = (x1 << jnp.uint32(r)) | (x1 >> jnp.uint32(32 - r))
            x1 = x1 ^ x0
        a, b = sched[i]
        x0 = x0 + a
        x1 = x1 + b + jnp.uint32(i + 1)
    return x0 ^ x1


def _gumbel_from_bits(bits):
    fb = (bits >> jnp.uint32(9)) | jnp.uint32(0x3F800000)
    f = jax.lax.bitcast_convert_type(fb, jnp.float32) - jnp.float32(1.0)
    u = jnp.maximum(_TINY, f * (jnp.float32(1.0) - _TINY) + _TINY)
    return -jnp.log(-jnp.log(u))


def _walks_kernel(src_ref, dst_ref, t_ref, n0_ref, t0_ref,
                  on_ref, ot_ref, om_ref):
    g = pl.program_id(0)
    cn = n0_ref[0]
    ct = t0_ref[0]
    alive = jnp.ones((8, 1), dtype=jnp.bool_)
    wk = jax.lax.broadcasted_iota(jnp.int32, (8, 1), 0) + 8 * g

    on_ref[0, :, 0:1] = cn
    ot_ref[0, :, 0:1] = ct
    om_ref[0, :, 0:1] = jnp.ones((8, 1), jnp.float32)

    for s in range(WALK_LEN - 1):
        k0, k1 = _STEP_KEYS[s]

        def scan_top2(cnx, ctx):
            rows = cnx.shape[0]

            def chunk_upd(off, bt1_, bt2_, bd_):
                src = src_ref[:, pl.ds(off, C1)]
                dst = dst_ref[:, pl.ds(off, C1)]
                tt = t_ref[:, pl.ds(off, C1)]
                valid = (src == cnx) & (tt < ctx)
                tc = jnp.where(valid, tt, _NEG_INF)
                cmax = jnp.max(tc, axis=1, keepdims=True)
                j = jax.lax.broadcasted_iota(jnp.int32, (1, C1), 1)
                jbig = jnp.where(tc == cmax, j, jnp.int32(2**31 - 1))
                jsel = jnp.min(jbig, axis=1, keepdims=True)
                onehot = j == jsel
                dsel = jnp.sum(jnp.where(onehot, dst, 0),
                               axis=1, keepdims=True)
                cmax2 = jnp.max(jnp.where(onehot, _NEG_INF, tc),
                                axis=1, keepdims=True)
                nb1 = jnp.maximum(bt1_, cmax)
                nb2 = jnp.maximum(jnp.minimum(bt1_, cmax),
                                  jnp.maximum(bt2_, cmax2))
                nbd = jnp.where(cmax > bt1_, dsel, bd_)
                return nb1, nb2, nbd

            neg = jnp.full((rows, 1), _NEG_INF, jnp.float32)
            zero = jnp.zeros((rows, 1), jnp.int32)
            acc = []
            for q in range(N1):
                acc.extend(chunk_upd(q * C1, neg, neg, zero))
            acc = tuple(acc)

            def merge(a1, a2, ad, b1, b2, bd_):
                m1 = jnp.maximum(a1, b1)
                m2 = jnp.maximum(jnp.minimum(a1, b1), jnp.maximum(a2, b2))
                md = jnp.where(a1 >= b1, ad, bd_)
                return m1, m2, md

            m = [merge(*acc[6 * q:6 * q + 6]) for q in range(N1 // 2)]
            while len(m) > 1:
                m = [merge(*m[2 * q], *m[2 * q + 1])
                     for q in range(len(m) // 2)]
            return m[0]

        if s == 0:
            cn2 = jnp.concatenate([cn[0:1], cn[7:8]], axis=0)
            ct2 = jnp.concatenate([ct[0:1], ct[7:8]], axis=0)
            b1_2, b2_2, bd_2 = scan_top2(cn2, ct2)
            sel = (wk // 10) == ((8 * g) // 10)
            tmax_raw = jnp.where(sel, b1_2[0:1], b1_2[1:2])
            t2_raw = jnp.where(sel, b2_2[0:1], b2_2[1:2])
            bd1 = jnp.where(sel, bd_2[0:1], bd_2[1:2])
        else:
            tmax_raw, t2_raw, bd1 = scan_top2(cn, ct)
        has_valid = tmax_raw > _NEG_INF
        tmax = jnp.where(has_valid, tmax_raw, jnp.float32(0.0))
        multi = jnp.any((t2_raw >= tmax_raw - jnp.float32(3.0))
                        & (t2_raw > _NEG_INF))

        def p2_body(i, carry):
            src = src_ref[:, pl.ds(i * C2, C2)]
            tt = t_ref[:, pl.ds(i * C2, C2)]
            valid = (src == cn) & (tt < ct)
            cand = valid & (tt >= tmax - jnp.float32(3.0))
            anyc = jnp.any(cand)

            def do_hash(carry):
                bv, bd, bt = carry
                dst = dst_ref[:, pl.ds(i * C2, C2)]
                j = i * C2 + jax.lax.broadcasted_iota(jnp.int32, (1, C2), 1)
                idxs = (wk * E_ALL + j).astype(jnp.uint32)
                gmb = _gumbel_from_bits(_tf_bits(idxs, k0, k1))
                logits = jnp.where(valid,
                                   (tt - tmax) / jnp.float32(TEMPERATURE),
                                   _NEG_INF)
                score = logits + gmb
                cmax = jnp.max(score, axis=1, keepdims=True)
                jbig = jnp.where(score == cmax, j, jnp.int32(2**31 - 1))
                jsel = jnp.min(jbig, axis=1, keepdims=True)
                onehot = j == jsel
                dsel = jnp.sum(jnp.where(onehot, dst, 0),
                               axis=1, keepdims=True)
                tsel = jnp.sum(jnp.where(onehot, tt, jnp.float32(0.0)),
                               axis=1, keepdims=True)
                upd = cmax > bv
                return (jnp.where(upd, cmax, bv),
                        jnp.where(upd, dsel, bd),
                        jnp.where(upd, tsel, bt))

            return jax.lax.cond(anyc, do_hash, lambda c: c, carry)

        def full_path():
            _, bd2, bt2 = jax.lax.fori_loop(
                0, N2, p2_body,
                (jnp.full((8, 1), _NEG_INF, jnp.float32),
                 jnp.zeros((8, 1), jnp.int32),
                 jnp.zeros((8, 1), jnp.float32)))
            return bd2, bt2

        def fast_path():
            return bd1, tmax_raw

        bd, bt = jax.lax.cond(multi, full_path, fast_path)

        alive = alive & has_valid
        cn = jnp.where(alive, bd, cn)
        ct = jnp.where(alive, bt, ct)
        on_ref[0, :, s + 1:s + 2] = jnp.where(alive, bd, 0)
        ot_ref[0, :, s + 1:s + 2] = jnp.where(alive, bt, jnp.float32(0.0))
        om_ref[0, :, s + 1:s + 2] = alive.astype(jnp.float32)


def _restart_kernel(sn_ref, ct_ref, mem_ref, w_ref, b_ref, tw_ref, tb_ref,
                    out_ref, mrows_ref):
    def gather_body(i, _):
        idx = sn_ref[i]
        mrows_ref[pl.ds(i, 1), :] = mem_ref[pl.ds(idx, 1), :]
        return 0

    jax.lax.fori_loop(0, BATCH, gather_body, 0)
    mem = mrows_ref[...]
    te = jnp.cos(ct_ref[...] * tw_ref[...] + tb_ref[...])
    wm = w_ref[:, :MEMORY_DIM]
    wt = w_ref[:, MEMORY_DIM:]
    r = (jnp.sum(mem * wm, axis=1, keepdims=True)
         + jnp.sum(te * wt, axis=1, keepdims=True) + b_ref[...])
    out_ref[...] = jax.nn.sigmoid(r)


def kernel(source_nodes, current_times, edge_index, edge_time, memory_states,
           W_restart, b_restart, time_w, time_b):
    src_all = jnp.concatenate([edge_index[0], edge_index[1]]).astype(jnp.int32)
    dst_all = jnp.concatenate([edge_index[1], edge_index[0]]).astype(jnp.int32)
    t_all = jnp.concatenate([edge_time, edge_time]).astype(jnp.float32)
    pad = E_PAD - E_ALL
    src_p = jnp.pad(src_all, (0, pad), constant_values=-1)[None, :]
    dst_p = jnp.pad(dst_all, (0, pad), constant_values=0)[None, :]
    t_p = jnp.pad(t_all, (0, pad), constant_values=0.0)[None, :]

    n0 = jnp.broadcast_to(source_nodes.astype(jnp.int32)[:, None],
                          (BATCH, NUM_WALKS)).reshape(GROUPS, 8, 1)
    t0 = jnp.broadcast_to(current_times.astype(jnp.float32)[:, None],
                          (BATCH, NUM_WALKS)).reshape(GROUPS, 8, 1)

    full = pl.BlockSpec((1, E_PAD), lambda g: (0, 0))
    state = pl.BlockSpec((1, 8, 1), lambda g: (g, 0, 0))
    out3 = pl.BlockSpec((1, 8, WALK_LEN), lambda g: (g, 0, 0))

    on, ot, om = pl.pallas_call(
        _walks_kernel,
        grid=(GROUPS,),
        in_specs=[full, full, full, state, state],
        out_specs=[out3, out3, out3],
        out_shape=[
            jax.ShapeDtypeStruct((GROUPS, 8, WALK_LEN), jnp.int32),
            jax.ShapeDtypeStruct((GROUPS, 8, WALK_LEN), jnp.float32),
            jax.ShapeDtypeStruct((GROUPS, 8, WALK_LEN), jnp.float32),
        ],
    )(src_p, dst_p, t_p, n0, t0)

    walk_nodes = on.reshape(BATCH, NUM_WALKS, WALK_LEN)
    walk_times = ot.reshape(BATCH, NUM_WALKS, WALK_LEN)
    walk_masks = om.reshape(BATCH, NUM_WALKS, WALK_LEN)

    restart_probs = pl.pallas_call(
        _restart_kernel,
        in_specs=[
            pl.BlockSpec(memory_space=pltpu.SMEM),
            pl.BlockSpec((BATCH, 1), lambda: (0, 0)),
            pl.BlockSpec((NUM_NODES, MEMORY_DIM), lambda: (0, 0)),
            pl.BlockSpec((1, MEMORY_DIM + TIME_DIM), lambda: (0, 0)),
            pl.BlockSpec((1, 1), lambda: (0, 0)),
            pl.BlockSpec((1, TIME_DIM), lambda: (0, 0)),
            pl.BlockSpec((1, TIME_DIM), lambda: (0, 0)),
        ],
        out_specs=pl.BlockSpec((BATCH, 1), lambda: (0, 0)),
        out_shape=jax.ShapeDtypeStruct((BATCH, 1), jnp.float32),
        scratch_shapes=[pltpu.VMEM((BATCH, MEMORY_DIM), jnp.float32)],
    )(source_nodes.astype(jnp.int32),
      current_times.astype(jnp.float32)[:, None],
      memory_states.astype(jnp.float32),
      W_restart.astype(jnp.float32).reshape(1, -1),
      b_restart.astype(jnp.float32).reshape(1, 1),
      time_w.astype(jnp.float32)[None, :],
      time_b.astype(jnp.float32)[None, :])

    return walk_nodes, walk_times, walk_masks, restart_probs


# speedup vs baseline: 38.4927x; 1.0007x over previous
import numpy as np
import jax
import jax.numpy as jnp
from jax.experimental import pallas as pl
from jax.experimental.pallas import tpu as pltpu

NUM_NODES = 10000
NUM_EDGES = 50000
BATCH = 32
MEMORY_DIM = 128
TIME_DIM = 64
NUM_WALKS = 10
WALK_LEN = 3
TEMPERATURE = 0.1

E_ALL = 2 * NUM_EDGES
W_TOT = BATCH * NUM_WALKS
GROUPS = W_TOT // 8
C1 = 6272
C2 = 1024
N1 = 16
N2 = 98
E_PAD = N1 * C1

_TINY = np.float32(np.finfo(np.float32).tiny)
_NEG_INF = np.float32(-np.inf)


def _np_threefry_pair(k0, k1, x0, x1):
    x0 = np.asarray(x0, np.uint32).copy()
    x1 = np.asarray(x1, np.uint32).copy()
    ks0 = np.uint32(k0)
    ks1 = np.uint32(k1)
    ks2 = np.uint32(ks0 ^ ks1 ^ np.uint32(0x1BD11BDA))
    rots = ((13, 15, 26, 6), (17, 29, 16, 24))
    sched = ((ks1, ks2), (ks2, ks0), (ks0, ks1), (ks1, ks2), (ks2, ks0))
    x0 = (x0 + ks0).astype(np.uint32)
    x1 = (x1 + ks1).astype(np.uint32)
    for i in range(5):
        for r in rots[i % 2]:
            x0 = (x0 + x1).astype(np.uint32)
            x1 = ((x1 << np.uint32(r)) | (x1 >> np.uint32(32 - r))).astype(np.uint32)
            x1 = (x1 ^ x0).astype(np.uint32)
        a, b = sched[i]
        x0 = (x0 + a).astype(np.uint32)
        x1 = (x1 + b + np.uint32(i + 1)).astype(np.uint32)
    return x0, x1


def _derive_step_keys():
    keys = []
    k = (np.uint32(0), np.uint32(1234))
    for _ in range(WALK_LEN - 1):
        h0, h1 = _np_threefry_pair(k[0], k[1], np.zeros(2, np.uint32),
                                   np.arange(2, dtype=np.uint32))
        k = (h0[0], h1[0])
        keys.append((int(h0[1]), int(h1[1])))
    return keys


_STEP_KEYS = _derive_step_keys()


def _tf_bits(x1, k0, k1):
    ks0 = jnp.uint32(k0)
    ks1 = jnp.uint32(k1)
    ks2 = jnp.uint32(k0 ^ k1 ^ 0x1BD11BDA)
    rots = ((13, 15, 26, 6), (17, 29, 16, 24))
    sched = ((ks1, ks2), (ks2, ks0), (ks0, ks1), (ks1, ks2), (ks2, ks0))
    x0 = jnp.full_like(x1, ks0)
    x1 = x1 + ks1
    for i in range(5):
        for r in rots[i % 2]:
            x0 = x0 + x1
            x1 = (x1 << jnp.uint32(r)) | (x1 >> jnp.uint32(32 - r))
            x1 = x1 ^ x0
        a, b = sched[i]
        x0 = x0 + a
        x1 = x1 + b + jnp.uint32(i + 1)
    return x0 ^ x1


def _gumbel_from_bits(bits):
    fb = (bits >> jnp.uint32(9)) | jnp.uint32(0x3F800000)
    f = jax.lax.bitcast_convert_type(fb, jnp.float32) - jnp.float32(1.0)
    u = jnp.maximum(_TINY, f * (jnp.float32(1.0) - _TINY) + _TINY)
    return -jnp.log(-jnp.log(u))


def _walks_kernel(src_ref, dst_ref, t_ref, n0_ref, t0_ref,
                  on_ref, ot_ref, om_ref):
    g = pl.program_id(0)
    cn = n0_ref[0]
    ct = t0_ref[0]
    alive = jnp.ones((8, 1), dtype=jnp.bool_)
    wk = jax.lax.broadcasted_iota(jnp.int32, (8, 1), 0) + 8 * g

    on_ref[0, :, 0:1] = cn
    ot_ref[0, :, 0:1] = ct
    om_ref[0, :, 0:1] = jnp.ones((8, 1), jnp.float32)

    for s in range(WALK_LEN - 1):
        k0, k1 = _STEP_KEYS[s]

        def scan_top2(cnx, ctx):
            rows = cnx.shape[0]

            def chunk_upd(off, bt1_, bt2_, bd_):
                src = src_ref[:, pl.ds(off, C1)]
                dst = dst_ref[:, pl.ds(off, C1)]
                tt = t_ref[:, pl.ds(off, C1)]
                valid = (src == cnx) & (tt < ctx)
                tc = jnp.where(valid, tt, _NEG_INF)
                cmax = jnp.max(tc, axis=1, keepdims=True)
                j = jax.lax.broadcasted_iota(jnp.int32, (1, C1), 1)
                jbig = jnp.where(tc == cmax, j, jnp.int32(2**31 - 1))
                jsel = jnp.min(jbig, axis=1, keepdims=True)
                onehot = j == jsel
                dsel = jnp.sum(jnp.where(onehot, dst, 0),
                               axis=1, keepdims=True)
                cmax2 = jnp.max(jnp.where(onehot, _NEG_INF, tc),
                                axis=1, keepdims=True)
                nb1 = jnp.maximum(bt1_, cmax)
                nb2 = jnp.maximum(jnp.minimum(bt1_, cmax),
                                  jnp.maximum(bt2_, cmax2))
                nbd = jnp.where(cmax > bt1_, dsel, bd_)
                return nb1, nb2, nbd

            neg = jnp.full((rows, 1), _NEG_INF, jnp.float32)
            zero = jnp.zeros((rows, 1), jnp.int32)
            acc = []
            for q in range(N1):
                acc.extend(chunk_upd(q * C1, neg, neg, zero))
            acc = tuple(acc)

            def merge(a1, a2, ad, b1, b2, bd_):
                m1 = jnp.maximum(a1, b1)
                m2 = jnp.maximum(jnp.minimum(a1, b1), jnp.maximum(a2, b2))
                md = jnp.where(a1 >= b1, ad, bd_)
                return m1, m2, md

            m = [merge(*acc[6 * q:6 * q + 6]) for q in range(N1 // 2)]
            while len(m) > 1:
                m = [merge(*m[2 * q], *m[2 * q + 1])
                     for q in range(len(m) // 2)]
            return m[0]

        if s == 0:
            cn2 = jnp.concatenate([cn[0:1], cn[7:8]], axis=0)
            ct2 = jnp.concatenate([ct[0:1], ct[7:8]], axis=0)
            b1_2, b2_2, bd_2 = scan_top2(cn2, ct2)
            sel = (wk // 10) == ((8 * g) // 10)
            tmax_raw = jnp.where(sel, b1_2[0:1], b1_2[1:2])
            t2_raw = jnp.where(sel, b2_2[0:1], b2_2[1:2])
            bd1 = jnp.where(sel, bd_2[0:1], bd_2[1:2])
        else:
            tmax_raw, t2_raw, bd1 = scan_top2(cn, ct)
        has_valid = tmax_raw > _NEG_INF
        tmax = jnp.where(has_valid, tmax_raw, jnp.float32(0.0))
        multi = jnp.any((t2_raw >= tmax_raw - jnp.float32(2.05))
                        & (t2_raw > _NEG_INF))

        def p2_body(i, carry):
            src = src_ref[:, pl.ds(i * C2, C2)]
            tt = t_ref[:, pl.ds(i * C2, C2)]
            valid = (src == cn) & (tt < ct)
            cand = valid & (tt >= tmax - jnp.float32(2.05))
            anyc = jnp.any(cand)

            def do_hash(carry):
                bv, bd, bt = carry
                dst = dst_ref[:, pl.ds(i * C2, C2)]
                j = i * C2 + jax.lax.broadcasted_iota(jnp.int32, (1, C2), 1)
                idxs = (wk * E_ALL + j).astype(jnp.uint32)
                gmb = _gumbel_from_bits(_tf_bits(idxs, k0, k1))
                logits = jnp.where(valid,
                                   (tt - tmax) / jnp.float32(TEMPERATURE),
                                   _NEG_INF)
                score = logits + gmb
                cmax = jnp.max(score, axis=1, keepdims=True)
                jbig = jnp.where(score == cmax, j, jnp.int32(2**31 - 1))
                jsel = jnp.min(jbig, axis=1, keepdims=True)
                onehot = j == jsel
                dsel = jnp.sum(jnp.where(onehot, dst, 0),
                               axis=1, keepdims=True)
                tsel = jnp.sum(jnp.where(onehot, tt, jnp.float32(0.0)),
                               axis=1, keepdims=True)
                upd = cmax > bv
                return (jnp.where(upd, cmax, bv),
                        jnp.where(upd, dsel, bd),
                        jnp.where(upd, tsel, bt))

            return jax.lax.cond(anyc, do_hash, lambda c: c, carry)

        def full_path():
            _, bd2, bt2 = jax.lax.fori_loop(
                0, N2, p2_body,
                (jnp.full((8, 1), _NEG_INF, jnp.float32),
                 jnp.zeros((8, 1), jnp.int32),
                 jnp.zeros((8, 1), jnp.float32)))
            return bd2, bt2

        def fast_path():
            return bd1, tmax_raw

        bd, bt = jax.lax.cond(multi, full_path, fast_path)

        alive = alive & has_valid
        cn = jnp.where(alive, bd, cn)
        ct = jnp.where(alive, bt, ct)
        on_ref[0, :, s + 1:s + 2] = jnp.where(alive, bd, 0)
        ot_ref[0, :, s + 1:s + 2] = jnp.where(alive, bt, jnp.float32(0.0))
        om_ref[0, :, s + 1:s + 2] = alive.astype(jnp.float32)


def _restart_kernel(sn_ref, ct_ref, mem_ref, w_ref, b_ref, tw_ref, tb_ref,
                    out_ref, mrows_ref):
    def gather_body(i, _):
        idx = sn_ref[i]
        mrows_ref[pl.ds(i, 1), :] = mem_ref[pl.ds(idx, 1), :]
        return 0

    jax.lax.fori_loop(0, BATCH, gather_body, 0)
    mem = mrows_ref[...]
    te = jnp.cos(ct_ref[...] * tw_ref[...] + tb_ref[...])
    wm = w_ref[:, :MEMORY_DIM]
    wt = w_ref[:, MEMORY_DIM:]
    r = (jnp.sum(mem * wm, axis=1, keepdims=True)
         + jnp.sum(te * wt, axis=1, keepdims=True) + b_ref[...])
    out_ref[...] = jax.nn.sigmoid(r)


def kernel(source_nodes, current_times, edge_index, edge_time, memory_states,
           W_restart, b_restart, time_w, time_b):
    src_all = jnp.concatenate([edge_index[0], edge_index[1]]).astype(jnp.int32)
    dst_all = jnp.concatenate([edge_index[1], edge_index[0]]).astype(jnp.int32)
    t_all = jnp.concatenate([edge_time, edge_time]).astype(jnp.float32)
    pad = E_PAD - E_ALL
    src_p = jnp.pad(src_all, (0, pad), constant_values=-1)[None, :]
    dst_p = jnp.pad(dst_all, (0, pad), constant_values=0)[None, :]
    t_p = jnp.pad(t_all, (0, pad), constant_values=0.0)[None, :]

    n0 = jnp.broadcast_to(source_nodes.astype(jnp.int32)[:, None],
                          (BATCH, NUM_WALKS)).reshape(GROUPS, 8, 1)
    t0 = jnp.broadcast_to(current_times.astype(jnp.float32)[:, None],
                          (BATCH, NUM_WALKS)).reshape(GROUPS, 8, 1)

    full = pl.BlockSpec((1, E_PAD), lambda g: (0, 0))
    state = pl.BlockSpec((1, 8, 1), lambda g: (g, 0, 0))
    out3 = pl.BlockSpec((1, 8, WALK_LEN), lambda g: (g, 0, 0))

    on, ot, om = pl.pallas_call(
        _walks_kernel,
        grid=(GROUPS,),
        in_specs=[full, full, full, state, state],
        out_specs=[out3, out3, out3],
        out_shape=[
            jax.ShapeDtypeStruct((GROUPS, 8, WALK_LEN), jnp.int32),
            jax.ShapeDtypeStruct((GROUPS, 8, WALK_LEN), jnp.float32),
            jax.ShapeDtypeStruct((GROUPS, 8, WALK_LEN), jnp.float32),
        ],
    )(src_p, dst_p, t_p, n0, t0)

    walk_nodes = on.reshape(BATCH, NUM_WALKS, WALK_LEN)
    walk_times = ot.reshape(BATCH, NUM_WALKS, WALK_LEN)
    walk_masks = om.reshape(BATCH, NUM_WALKS, WALK_LEN)

    restart_probs = pl.pallas_call(
        _restart_kernel,
        in_specs=[
            pl.BlockSpec(memory_space=pltpu.SMEM),
            pl.BlockSpec((BATCH, 1), lambda: (0, 0)),
            pl.BlockSpec((NUM_NODES, MEMORY_DIM), lambda: (0, 0)),
            pl.BlockSpec((1, MEMORY_DIM + TIME_DIM), lambda: (0, 0)),
            pl.BlockSpec((1, 1), lambda: (0, 0)),
            pl.BlockSpec((1, TIME_DIM), lambda: (0, 0)),
            pl.BlockSpec((1, TIME_DIM), lambda: (0, 0)),
        ],
        out_specs=pl.BlockSpec((BATCH, 1), lambda: (0, 0)),
        out_shape=jax.ShapeDtypeStruct((BATCH, 1), jnp.float32),
        scratch_shapes=[pltpu.VMEM((BATCH, MEMORY_DIM), jnp.float32)],
    )(source_nodes.astype(jnp.int32),
      current_times.astype(jnp.float32)[:, None],
      memory_states.astype(jnp.float32),
      W_restart.astype(jnp.float32).reshape(1, -1),
      b_restart.astype(jnp.float32).reshape(1, 1),
      time_w.astype(jnp.float32)[None, :],
      time_b.astype(jnp.float32)[None, :])

    return walk_nodes, walk_times, walk_masks, restart_probs


# speedup vs baseline: 53.1803x; 1.3816x over previous
import numpy as np
import jax
import jax.numpy as jnp
from jax.experimental import pallas as pl
from jax.experimental.pallas import tpu as pltpu

NUM_NODES = 10000
NUM_EDGES = 50000
BATCH = 32
MEMORY_DIM = 128
TIME_DIM = 64
NUM_WALKS = 10
WALK_LEN = 3
TEMPERATURE = 0.1

E_ALL = 2 * NUM_EDGES
W_TOT = BATCH * NUM_WALKS
GROUPS = W_TOT // 8
C1 = 3200
C2 = 1024
N1 = 16
N2 = 50
E_PAD = N1 * C1
_BIG = np.int32(2**31 - 1)

_TINY = np.float32(np.finfo(np.float32).tiny)
_NEG_INF = np.float32(-np.inf)


def _np_threefry_pair(k0, k1, x0, x1):
    x0 = np.asarray(x0, np.uint32).copy()
    x1 = np.asarray(x1, np.uint32).copy()
    ks0 = np.uint32(k0)
    ks1 = np.uint32(k1)
    ks2 = np.uint32(ks0 ^ ks1 ^ np.uint32(0x1BD11BDA))
    rots = ((13, 15, 26, 6), (17, 29, 16, 24))
    sched = ((ks1, ks2), (ks2, ks0), (ks0, ks1), (ks1, ks2), (ks2, ks0))
    x0 = (x0 + ks0).astype(np.uint32)
    x1 = (x1 + ks1).astype(np.uint32)
    for i in range(5):
        for r in rots[i % 2]:
            x0 = (x0 + x1).astype(np.uint32)
            x1 = ((x1 << np.uint32(r)) | (x1 >> np.uint32(32 - r))).astype(np.uint32)
            x1 = (x1 ^ x0).astype(np.uint32)
        a, b = sched[i]
        x0 = (x0 + a).astype(np.uint32)
        x1 = (x1 + b + np.uint32(i + 1)).astype(np.uint32)
    return x0, x1


def _derive_step_keys():
    keys = []
    k = (np.uint32(0), np.uint32(1234))
    for _ in range(WALK_LEN - 1):
        h0, h1 = _np_threefry_pair(k[0], k[1], np.zeros(2, np.uint32),
                                   np.arange(2, dtype=np.uint32))
        k = (h0[0], h1[0])
        keys.append((int(h0[1]), int(h1[1])))
    return keys


_STEP_KEYS = _derive_step_keys()


def _tf_bits(x1, k0, k1):
    ks0 = jnp.uint32(k0)
    ks1 = jnp.uint32(k1)
    ks2 = jnp.uint32(k0 ^ k1 ^ 0x1BD11BDA)
    rots = ((13, 15, 26, 6), (17, 29, 16, 24))
    sched = ((ks1, ks2), (ks2, ks0), (ks0, ks1), (ks1, ks2), (ks2, ks0))
    x0 = jnp.full_like(x1, ks0)
    x1 = x1 + ks1
    for i in range(5):
        for r in rots[i % 2]:
            x0 = x0 + x1
            x1 = (x1 << jnp.uint32(r)) | (x1 >> jnp.uint32(32 - r))
            x1 = x1 ^ x0
        a, b = sched[i]
        x0 = x0 + a
        x1 = x1 + b + jnp.uint32(i + 1)
    return x0 ^ x1


def _gumbel_from_bits(bits):
    fb = (bits >> jnp.uint32(9)) | jnp.uint32(0x3F800000)
    f = jax.lax.bitcast_convert_type(fb, jnp.float32) - jnp.float32(1.0)
    u = jnp.maximum(_TINY, f * (jnp.float32(1.0) - _TINY) + _TINY)
    return -jnp.log(-jnp.log(u))


def _walks_kernel(u_ref, v_ref, t_ref, n0_ref, t0_ref,
                  on_ref, ot_ref, om_ref):
    g = pl.program_id(0)
    cn = n0_ref[0]
    ct = t0_ref[0]
    alive = jnp.ones((8, 1), dtype=jnp.bool_)
    wk = jax.lax.broadcasted_iota(jnp.int32, (8, 1), 0) + 8 * g

    on_ref[0, :, 0:1] = cn
    ot_ref[0, :, 0:1] = ct
    om_ref[0, :, 0:1] = jnp.ones((8, 1), jnp.float32)

    for s in range(WALK_LEN - 1):
        k0, k1 = _STEP_KEYS[s]

        def scan_top2(cnx, ctx):
            rows = cnx.shape[0]

            def chunk_upd(off, bt1_, bt2_, bd_):
                u = u_ref[:, pl.ds(off, C1)]
                v = v_ref[:, pl.ds(off, C1)]
                tt = t_ref[:, pl.ds(off, C1)]
                mf = u == cnx
                valid = (mf | (v == cnx)) & (tt < ctx)
                tc = jnp.where(valid, tt, _NEG_INF)
                cmax = jnp.max(tc, axis=1, keepdims=True)
                j = jax.lax.broadcasted_iota(jnp.int32, (1, C1), 1)
                jbig = jnp.where(tc == cmax, j, _BIG)
                jsel = jnp.min(jbig, axis=1, keepdims=True)
                onehot = j == jsel
                dboth = jnp.where(mf, v, u)
                dsel = jnp.sum(jnp.where(onehot, dboth, 0),
                               axis=1, keepdims=True)
                cmax2 = jnp.max(jnp.where(onehot, _NEG_INF, tc),
                                axis=1, keepdims=True)
                nb1 = jnp.maximum(bt1_, cmax)
                nb2 = jnp.maximum(jnp.minimum(bt1_, cmax),
                                  jnp.maximum(bt2_, cmax2))
                nbd = jnp.where(cmax > bt1_, dsel, bd_)
                return nb1, nb2, nbd

            neg = jnp.full((rows, 1), _NEG_INF, jnp.float32)
            zero = jnp.zeros((rows, 1), jnp.int32)
            acc = []
            for q in range(N1):
                acc.extend(chunk_upd(q * C1, neg, neg, zero))
            acc = tuple(acc)

            def merge(a1, a2, ad, b1, b2, bd_):
                m1 = jnp.maximum(a1, b1)
                m2 = jnp.maximum(jnp.minimum(a1, b1), jnp.maximum(a2, b2))
                md = jnp.where(a1 >= b1, ad, bd_)
                return m1, m2, md

            m = [merge(*acc[6 * q:6 * q + 6]) for q in range(N1 // 2)]
            while len(m) > 1:
                m = [merge(*m[2 * q], *m[2 * q + 1])
                     for q in range(len(m) // 2)]
            return m[0]

        if s == 0:
            cn2 = jnp.concatenate([cn[0:1], cn[7:8]], axis=0)
            ct2 = jnp.concatenate([ct[0:1], ct[7:8]], axis=0)
            b1_2, b2_2, bd_2 = scan_top2(cn2, ct2)
            sel = (wk // 10) == ((8 * g) // 10)
            tmax_raw = jnp.where(sel, b1_2[0:1], b1_2[1:2])
            t2_raw = jnp.where(sel, b2_2[0:1], b2_2[1:2])
            bd1 = jnp.where(sel, bd_2[0:1], bd_2[1:2])
        else:
            tmax_raw, t2_raw, bd1 = scan_top2(cn, ct)
        has_valid = tmax_raw > _NEG_INF
        tmax = jnp.where(has_valid, tmax_raw, jnp.float32(0.0))
        multi = jnp.any((t2_raw >= tmax_raw - jnp.float32(2.05))
                        & (t2_raw > _NEG_INF))

        def p2_body(i, carry):
            u = u_ref[:, pl.ds(i * C2, C2)]
            v = v_ref[:, pl.ds(i * C2, C2)]
            tt = t_ref[:, pl.ds(i * C2, C2)]
            mf = u == cn
            mb = v == cn
            tv = tt < ct
            valid = (mf | mb) & tv
            cand = valid & (tt >= tmax - jnp.float32(2.05))
            anyc = jnp.any(cand)

            def do_hash(carry):
                bv, bd, bt, bj = carry
                jf = i * C2 + jax.lax.broadcasted_iota(jnp.int32, (1, C2), 1)
                base = wk * E_ALL + jf
                gf = _gumbel_from_bits(
                    _tf_bits(base.astype(jnp.uint32), k0, k1))
                gb = _gumbel_from_bits(
                    _tf_bits((base + NUM_EDGES).astype(jnp.uint32), k0, k1))
                logit = (tt - tmax) / jnp.float32(TEMPERATURE)
                sf = jnp.where(mf & tv, logit + gf, _NEG_INF)
                sb = jnp.where(mb & tv, logit + gb, _NEG_INF)
                pf = sf >= sb
                sc = jnp.maximum(sf, sb)
                je = jnp.where(pf, jf, jf + NUM_EDGES)
                de = jnp.where(pf, v, u)
                cmax = jnp.max(sc, axis=1, keepdims=True)
                jwh = jnp.where(sc == cmax, je, _BIG)
                jsel = jnp.min(jwh, axis=1, keepdims=True)
                onehot = jwh == jsel
                dsel = jnp.sum(jnp.where(onehot, de, 0),
                               axis=1, keepdims=True)
                tsel = jnp.sum(jnp.where(onehot, tt, jnp.float32(0.0)),
                               axis=1, keepdims=True)
                upd = (cmax > bv) | ((cmax == bv) & (jsel < bj))
                return (jnp.where(upd, cmax, bv),
                        jnp.where(upd, dsel, bd),
                        jnp.where(upd, tsel, bt),
                        jnp.where(upd, jsel, bj))

            return jax.lax.cond(anyc, do_hash, lambda c: c, carry)

        def full_path():
            _, bd2, bt2, _ = jax.lax.fori_loop(
                0, N2, p2_body,
                (jnp.full((8, 1), _NEG_INF, jnp.float32),
                 jnp.zeros((8, 1), jnp.int32),
                 jnp.zeros((8, 1), jnp.float32),
                 jnp.full((8, 1), _BIG, jnp.int32)))
            return bd2, bt2

        def fast_path():
            return bd1, tmax_raw

        bd, bt = jax.lax.cond(multi, full_path, fast_path)

        alive = alive & has_valid
        cn = jnp.where(alive, bd, cn)
        ct = jnp.where(alive, bt, ct)
        on_ref[0, :, s + 1:s + 2] = jnp.where(alive, bd, 0)
        ot_ref[0, :, s + 1:s + 2] = jnp.where(alive, bt, jnp.float32(0.0))
        om_ref[0, :, s + 1:s + 2] = alive.astype(jnp.float32)


def _restart_kernel(sn_ref, ct_ref, mem_ref, w_ref, b_ref, tw_ref, tb_ref,
                    out_ref, mrows_ref):
    def gather_body(i, _):
        idx = sn_ref[i]
        mrows_ref[pl.ds(i, 1), :] = mem_ref[pl.ds(idx, 1), :]
        return 0

    jax.lax.fori_loop(0, BATCH, gather_body, 0)
    mem = mrows_ref[...]
    te = jnp.cos(ct_ref[...] * tw_ref[...] + tb_ref[...])
    wm = w_ref[:, :MEMORY_DIM]
    wt = w_ref[:, MEMORY_DIM:]
    r = (jnp.sum(mem * wm, axis=1, keepdims=True)
         + jnp.sum(te * wt, axis=1, keepdims=True) + b_ref[...])
    out_ref[...] = jax.nn.sigmoid(r)


def kernel(source_nodes, current_times, edge_index, edge_time, memory_states,
           W_restart, b_restart, time_w, time_b):
    pad = E_PAD - NUM_EDGES
    u_p = jnp.pad(edge_index[0].astype(jnp.int32), (0, pad),
                  constant_values=-1)[None, :]
    v_p = jnp.pad(edge_index[1].astype(jnp.int32), (0, pad),
                  constant_values=-1)[None, :]
    t_p = jnp.pad(edge_time.astype(jnp.float32), (0, pad),
                  constant_values=0.0)[None, :]

    n0 = jnp.broadcast_to(source_nodes.astype(jnp.int32)[:, None],
                          (BATCH, NUM_WALKS)).reshape(GROUPS, 8, 1)
    t0 = jnp.broadcast_to(current_times.astype(jnp.float32)[:, None],
                          (BATCH, NUM_WALKS)).reshape(GROUPS, 8, 1)

    full = pl.BlockSpec((1, E_PAD), lambda g: (0, 0))
    state = pl.BlockSpec((1, 8, 1), lambda g: (g, 0, 0))
    out3 = pl.BlockSpec((1, 8, WALK_LEN), lambda g: (g, 0, 0))

    on, ot, om = pl.pallas_call(
        _walks_kernel,
        grid=(GROUPS,),
        in_specs=[full, full, full, state, state],
        out_specs=[out3, out3, out3],
        out_shape=[
            jax.ShapeDtypeStruct((GROUPS, 8, WALK_LEN), jnp.int32),
            jax.ShapeDtypeStruct((GROUPS, 8, WALK_LEN), jnp.float32),
            jax.ShapeDtypeStruct((GROUPS, 8, WALK_LEN), jnp.float32),
        ],
    )(u_p, v_p, t_p, n0, t0)

    walk_nodes = on.reshape(BATCH, NUM_WALKS, WALK_LEN)
    walk_times = ot.reshape(BATCH, NUM_WALKS, WALK_LEN)
    walk_masks = om.reshape(BATCH, NUM_WALKS, WALK_LEN)

    restart_probs = pl.pallas_call(
        _restart_kernel,
        in_specs=[
            pl.BlockSpec(memory_space=pltpu.SMEM),
            pl.BlockSpec((BATCH, 1), lambda: (0, 0)),
            pl.BlockSpec((NUM_NODES, MEMORY_DIM), lambda: (0, 0)),
            pl.BlockSpec((1, MEMORY_DIM + TIME_DIM), lambda: (0, 0)),
            pl.BlockSpec((1, 1), lambda: (0, 0)),
            pl.BlockSpec((1, TIME_DIM), lambda: (0, 0)),
            pl.BlockSpec((1, TIME_DIM), lambda: (0, 0)),
        ],
        out_specs=pl.BlockSpec((BATCH, 1), lambda: (0, 0)),
        out_shape=jax.ShapeDtypeStruct((BATCH, 1), jnp.float32),
        scratch_shapes=[pltpu.VMEM((BATCH, MEMORY_DIM), jnp.float32)],
    )(source_nodes.astype(jnp.int32),
      current_times.astype(jnp.float32)[:, None],
      memory_states.astype(jnp.float32),
      W_restart.astype(jnp.float32).reshape(1, -1),
      b_restart.astype(jnp.float32).reshape(1, 1),
      time_w.astype(jnp.float32)[None, :],
      time_b.astype(jnp.float32)[None, :])

    return walk_nodes, walk_times, walk_masks, restart_probs
